# baseline (speedup 1.0000x reference)
import functools

import jax
import jax.numpy as jnp
from jax import lax
from jax.experimental import pallas as pl
from jax.experimental.pallas import tpu as pltpu
from jax.experimental.pallas import tpu_sc as plsc

NC = 2
NS = 16
LANES = 16
NW = NC * NS
CHUNK = 128
NB = 8


def _zero_rows(buf, nrows, d):
    def body(i, _):
        for k in range(d // LANES):
            buf[i, pl.ds(k * LANES, LANES)] = jnp.zeros((LANES,), jnp.float32)
        return 0
    lax.fori_loop(0, nrows, body, 0)


def _hop_edges(ytab_sh, acc_sh, rowi_hbm, coli_hbm, out_hbm, idxr_v, idxc_v,
               rows_v, sems, c, s, m0, m1, mmax, n_pad, d):
    rows_per_sub = n_pad // NS
    grp = rows_per_sub // CHUNK
    m = jnp.where(c == 0, m0, m1)
    base = jnp.where(c == 0, s * m0, NS * m0 + s * m1)
    pltpu.sync_copy(rowi_hbm.at[pl.ds(base, mmax)], idxr_v)
    pltpu.sync_copy(coli_hbm.at[pl.ds(base, mmax)], idxc_v)
    _zero_rows(rows_v.at[0], CHUNK, d)
    for t in range(grp):
        pltpu.sync_copy(rows_v.at[0],
                        acc_sh.at[pl.ds(s * rows_per_sub + t * CHUNK, CHUNK)])
    plsc.subcore_barrier()

    for b in range(NB):
        pltpu.async_copy(ytab_sh.at[idxr_v.at[b]], rows_v.at[b], sems[b])

    @pl.loop(0, mmax, step=NB)
    def _chunks(g):
        @pl.when(g < m)
        def _():
            for b in range(NB):
                j = g + b
                pltpu.make_async_copy(
                    ytab_sh.at[idxr_v.at[j]], rows_v.at[b], sems[b]).wait()
                pltpu.sync_copy(rows_v.at[b], acc_sh.at[idxc_v.at[j]],
                                add=True)
                jn = j + NB

                @pl.when(jn < m)
                def _():
                    pltpu.async_copy(ytab_sh.at[idxr_v.at[jn]],
                                     rows_v.at[b], sems[b])

    plsc.subcore_barrier()
    for t in range(grp):
        off = s * rows_per_sub + t * CHUNK
        pltpu.sync_copy(acc_sh.at[pl.ds(off, CHUNK)],
                        out_hbm.at[c, pl.ds(off, CHUNK)])


def _hop_scratch(n_pad, d, mmax):
    return [
        pltpu.VMEM((mmax, CHUNK), jnp.int32),
        pltpu.VMEM((mmax, CHUNK), jnp.int32),
        pltpu.VMEM((NB, CHUNK, d), jnp.float32),
        pltpu.VMEM_SHARED((n_pad, d), jnp.float32),
        pltpu.VMEM_SHARED((n_pad, d), jnp.float32),
    ] + [pltpu.SemaphoreType.DMA] * NB


_MESH = dict(core_axis_name="c", subcore_axis_name="s")


def _make_hop_first(n_pad, d, m0, m1):
    mmax = max(m0, m1)
    rows_per_sub = n_pad // NS

    @functools.partial(
        pl.kernel, mesh=plsc.VectorSubcoreMesh(**_MESH),
        out_type=jax.ShapeDtypeStruct((NC, n_pad, d), jnp.float32),
        compiler_params=pltpu.CompilerParams(use_tc_tiling_on_sc=False),
        scratch_types=_hop_scratch(n_pad, d, mmax))
    def hop(y_hbm, rowi_hbm, coli_hbm, out_hbm, idxr_v, idxc_v, rows_v,
            ytab_sh, acc_sh, *sems):
        c = lax.axis_index("c")
        s = lax.axis_index("s")
        off = s * rows_per_sub
        pltpu.sync_copy(y_hbm.at[pl.ds(off, rows_per_sub)],
                        ytab_sh.at[pl.ds(off, rows_per_sub)])
        _hop_edges(ytab_sh, acc_sh, rowi_hbm, coli_hbm, out_hbm, idxr_v,
                   idxc_v, rows_v, sems, c, s, m0, m1, mmax, n_pad, d)

    return hop


def _make_hop_mid(n_pad, d, m0, m1, lo_col):
    mmax = max(m0, m1)
    rows_per_sub = n_pad // NS
    grp = rows_per_sub // CHUNK

    @functools.partial(
        pl.kernel, mesh=plsc.VectorSubcoreMesh(**_MESH),
        out_type=jax.ShapeDtypeStruct((NC, n_pad, d), jnp.float32),
        compiler_params=pltpu.CompilerParams(use_tc_tiling_on_sc=False),
        scratch_types=[
            pltpu.VMEM((CHUNK, d), jnp.float32),
            pltpu.VMEM((CHUNK, d), jnp.float32),
            pltpu.VMEM((CHUNK, LANES), jnp.float32),
            pltpu.VMEM((CHUNK, d), jnp.float32),
        ] + _hop_scratch(n_pad, d, mmax))
    def hop(p_hbm, invd_hbm, rowi_hbm, coli_hbm, out_hbm, sa, sb, sd, yb,
            idxr_v, idxc_v, rows_v, ytab_sh, acc_sh, *sems):
        c = lax.axis_index("c")
        s = lax.axis_index("s")
        for t in range(grp):
            off = s * rows_per_sub + t * CHUNK
            pltpu.sync_copy(
                p_hbm.at[0, pl.ds(off, CHUNK), pl.ds(lo_col, d)], sa)
            pltpu.sync_copy(
                p_hbm.at[1, pl.ds(off, CHUNK), pl.ds(lo_col, d)], sb)
            pltpu.sync_copy(invd_hbm.at[pl.ds(off, CHUNK)], sd)

            def rowbody(i, _):
                for k in range(d // LANES):
                    sl = pl.ds(k * LANES, LANES)
                    yb[i, sl] = (sa[i, sl] + sb[i, sl]) * sd[i, :]
                return 0
            lax.fori_loop(0, CHUNK, rowbody, 0)
            pltpu.sync_copy(yb, ytab_sh.at[pl.ds(off, CHUNK)])
        _hop_edges(ytab_sh, acc_sh, rowi_hbm, coli_hbm, out_hbm, idxr_v,
                   idxc_v, rows_v, sems, c, s, m0, m1, mmax, n_pad, d)

    return hop


def _make_hop_relu(n_pad, d, m0, m1):
    mmax = max(m0, m1)
    rows_per_sub = n_pad // NS
    grp = rows_per_sub // CHUNK

    @functools.partial(
        pl.kernel, mesh=plsc.VectorSubcoreMesh(**_MESH),
        out_type=(jax.ShapeDtypeStruct((NC, n_pad, d), jnp.float32),
                  jax.ShapeDtypeStruct((n_pad, d), jnp.float32)),
        compiler_params=pltpu.CompilerParams(use_tc_tiling_on_sc=False),
        scratch_types=[
            pltpu.VMEM((CHUNK, d), jnp.float32),
            pltpu.VMEM((CHUNK, d), jnp.float32),
            pltpu.VMEM((CHUNK, d), jnp.float32),
            pltpu.VMEM((CHUNK, d), jnp.float32),
            pltpu.VMEM((CHUNK, d), jnp.float32),
            pltpu.VMEM((CHUNK, LANES), jnp.float32),
            pltpu.VMEM((CHUNK, d), jnp.float32),
            pltpu.VMEM((CHUNK, d), jnp.float32),
        ] + _hop_scratch(n_pad, d, mmax))
    def hop(pab_hbm, pc_hbm, xw_hbm, dinv_hbm, rowi_hbm, coli_hbm,
            out_hbm, h_hbm, sa, sb, sc0, sc1, sx, sd, yb, hb,
            idxr_v, idxc_v, rows_v, ytab_sh, acc_sh, *sems):
        c = lax.axis_index("c")
        s = lax.axis_index("s")
        for t in range(grp):
            off = s * rows_per_sub + t * CHUNK
            pltpu.sync_copy(pab_hbm.at[0, pl.ds(off, CHUNK), pl.ds(0, d)], sa)
            pltpu.sync_copy(pab_hbm.at[1, pl.ds(off, CHUNK), pl.ds(0, d)], sb)
            pltpu.sync_copy(pc_hbm.at[0, pl.ds(off, CHUNK)], sc0)
            pltpu.sync_copy(pc_hbm.at[1, pl.ds(off, CHUNK)], sc1)
            pltpu.sync_copy(xw_hbm.at[pl.ds(off, CHUNK)], sx)
            pltpu.sync_copy(dinv_hbm.at[pl.ds(off, CHUNK)], sd)

            def rowbody(i, _):
                for k in range(d // LANES):
                    sl = pl.ds(k * LANES, LANES)
                    z = sa[i, sl] + sb[i, sl] + sc0[i, sl] + sc1[i, sl]
                    h = jnp.maximum(sx[i, sl] + z * sd[i, :], 0.0)
                    hb[i, sl] = h
                    yb[i, sl] = h * sd[i, :]
                return 0
            lax.fori_loop(0, CHUNK, rowbody, 0)
            pltpu.sync_copy(hb, h_hbm.at[pl.ds(off, CHUNK)])
            pltpu.sync_copy(yb, ytab_sh.at[pl.ds(off, CHUNK)])
        _hop_edges(ytab_sh, acc_sh, rowi_hbm, coli_hbm, out_hbm, idxr_v,
                   idxc_v, rows_v, sems, c, s, m0, m1, mmax, n_pad, d)

    return hop


def _make_deg(n_pad, n_chunks):
    rows_per_sub = n_pad // NS
    grp = rows_per_sub // CHUNK

    @functools.partial(
        pl.kernel, mesh=plsc.VectorSubcoreMesh(**_MESH),
        out_type=jax.ShapeDtypeStruct((NC, n_pad, LANES), jnp.float32),
        compiler_params=pltpu.CompilerParams(use_tc_tiling_on_sc=False),
        scratch_types=[
            pltpu.VMEM((n_chunks, CHUNK), jnp.int32),
            pltpu.VMEM((CHUNK, LANES), jnp.float32),
            pltpu.VMEM_SHARED((n_pad, LANES), jnp.float32),
        ])
    def deg(coli_hbm, out_hbm, idxc_v, ones_v, acc_sh):
        c = lax.axis_index("c")
        s = lax.axis_index("s")
        wid = c * NS + s
        pltpu.sync_copy(coli_hbm.at[pl.ds(wid * n_chunks, n_chunks)], idxc_v)
        _zero_rows(ones_v, CHUNK, LANES)
        for t in range(grp):
            pltpu.sync_copy(
                ones_v, acc_sh.at[pl.ds(s * rows_per_sub + t * CHUNK, CHUNK)])
        def fill(i, _):
            ones_v[i, pl.ds(0, LANES)] = jnp.ones((LANES,), jnp.float32)
            return 0
        lax.fori_loop(0, CHUNK, fill, 0)
        plsc.subcore_barrier()

        def body(j, _):
            pltpu.sync_copy(ones_v, acc_sh.at[idxc_v.at[j]], add=True)
            return 0
        lax.fori_loop(0, n_chunks, body, 0)
        plsc.subcore_barrier()
        for t in range(grp):
            off = s * rows_per_sub + t * CHUNK
            pltpu.sync_copy(acc_sh.at[pl.ds(off, CHUNK)],
                            out_hbm.at[c, pl.ds(off, CHUNK)])

    return deg


_BLK = 1024


def _proj_body(degp_ref, x_ref, w_ref, b_ref, xw_ref, y12_ref, dinv_ref,
               invd_ref):
    dsum = degp_ref[0] + degp_ref[1]
    pos = dsum > 0
    dinv = jnp.where(pos, lax.rsqrt(dsum), 0.0)
    dinv_ref[...] = dinv
    invd_ref[...] = jnp.where(pos, 1.0 / dsum, 0.0)
    d1 = dinv[:, 0:1]
    xb = x_ref[...]
    dd = xb.shape[1]
    hdim = xw_ref.shape[1]
    xw_ref[...] = (
        jnp.dot(xb, w_ref[0:dd], preferred_element_type=jnp.float32)
        + b_ref[...])
    u1 = jnp.dot(xb, w_ref[dd:2 * dd], preferred_element_type=jnp.float32)
    u2 = jnp.dot(xb, w_ref[2 * dd:3 * dd], preferred_element_type=jnp.float32)
    y12_ref[:, 0:hdim] = u1 * d1
    y12_ref[:, hdim:2 * hdim] = u2 * d1


def _out_body(h_ref, qa_ref, qb_ref, dinv_ref, w_ref, b_ref, out_ref):
    d1 = dinv_ref[...][:, 0:1]
    x1 = (qa_ref[0] + qa_ref[1]) * d1
    x2 = (qb_ref[0] + qb_ref[1]) * d1
    hh = h_ref.shape[1]
    acc = jnp.dot(h_ref[...], w_ref[0:hh], preferred_element_type=jnp.float32)
    acc += jnp.dot(x1, w_ref[hh:2 * hh], preferred_element_type=jnp.float32)
    acc += jnp.dot(x2, w_ref[2 * hh:3 * hh], preferred_element_type=jnp.float32)
    out_ref[...] = acc + b_ref[...]


def _row_spec(d):
    return pl.BlockSpec((_BLK, d), lambda i: (i, 0))


def _pair_spec(d):
    return pl.BlockSpec((NC, _BLK, d), lambda i: (0, i, 0))


def _full_spec(shape):
    return pl.BlockSpec(shape, lambda i: tuple(0 for _ in shape))


def kernel(x, edge_index, W1, b1, W2, b2):
    n, dd = x.shape
    hdim = W1.shape[1]
    e = edge_index.shape[1]

    n_pad = -(-n // (NS * CHUNK)) * (NS * CHUNK)
    e_pad = -(-e // (NW * CHUNK * NB)) * (NW * CHUNK * NB)
    n_chunks = e_pad // (NW * CHUNK)
    mm = 2 * n_chunks
    m0_32, m1_32 = 80, 80
    m0_16, m1_16 = 80, 80
    assert m0_32 + m1_32 == mm and m0_16 + m1_16 == mm
    padc = max(m0_32, m0_16, m1_32, m1_16)
    grid = n_pad // _BLK

    row = jnp.pad(edge_index[0], (0, e_pad - e))
    col = jnp.pad(edge_index[1], (0, e_pad - e),
                  constant_values=n)
    rowi = jnp.pad(row.reshape(NW * n_chunks, CHUNK), ((0, padc), (0, 0)))
    coli = jnp.pad(col.reshape(NW * n_chunks, CHUNK), ((0, padc), (0, 0)),
                   constant_values=n)
    x_pad = jnp.pad(x, ((0, n_pad - n), (0, 0)))

    hopA = _make_hop_first(n_pad, 2 * hdim, m0_32, m1_32)
    hopC = _make_hop_mid(n_pad, hdim, m0_16, m1_16, hdim)
    hopD = _make_hop_relu(n_pad, hdim, m0_16, m1_16)
    hopE = _make_hop_mid(n_pad, hdim, m0_16, m1_16, 0)
    degk = _make_deg(n_pad, n_chunks)

    degp = degk(coli)

    xw, y12, dinv, invd = pl.pallas_call(
        _proj_body,
        grid=(grid,),
        in_specs=[_pair_spec(LANES), _row_spec(dd), _full_spec(W1.shape),
                  _full_spec((1, hdim))],
        out_specs=[_row_spec(hdim), _row_spec(2 * hdim), _row_spec(LANES),
                   _row_spec(LANES)],
        out_shape=[jax.ShapeDtypeStruct((n_pad, hdim), jnp.float32),
                   jax.ShapeDtypeStruct((n_pad, 2 * hdim), jnp.float32),
                   jax.ShapeDtypeStruct((n_pad, LANES), jnp.float32),
                   jax.ShapeDtypeStruct((n_pad, LANES), jnp.float32)],
    )(degp, x_pad, W1, b1.reshape(1, hdim))

    Pab = hopA(y12, rowi, coli)
    Pc = hopC(Pab, invd, rowi, coli)
    Q1, h = hopD(Pab, Pc, xw, dinv, rowi, coli)
    Q2 = hopE(Q1, invd, rowi, coli)

    out = pl.pallas_call(
        _out_body,
        grid=(grid,),
        in_specs=[_row_spec(hdim), _pair_spec(hdim), _pair_spec(hdim),
                  _row_spec(LANES), _full_spec(W2.shape), _full_spec((1, dd))],
        out_specs=_row_spec(dd),
        out_shape=jax.ShapeDtypeStruct((n_pad, dd), jnp.float32),
    )(h, Q1, Q2, dinv, W2, b2.reshape(1, dd))

    return out[:n]


# speedup vs baseline: 1.0242x; 1.0242x over previous
import functools

import jax
import jax.numpy as jnp
from jax import lax
from jax.experimental import pallas as pl
from jax.experimental.pallas import tpu as pltpu
from jax.experimental.pallas import tpu_sc as plsc

NC = 2
NS = 16
LANES = 16
NW = NC * NS
CHUNK = 128
NB = 4
NBUF = 2 * NB


def _zero_rows(buf, nrows, d):
    def body(i, _):
        for k in range(d // LANES):
            buf[i, pl.ds(k * LANES, LANES)] = jnp.zeros((LANES,), jnp.float32)
        return 0
    lax.fori_loop(0, nrows, body, 0)


def _hop_edges(ytab_sh, acc_sh, rowi_hbm, coli_hbm, out_hbm, idxr_v, idxc_v,
               rows_v, sems, c, s, m0, m1, mmax, n_pad, d):
    rows_per_sub = n_pad // NS
    grp = rows_per_sub // CHUNK
    m = jnp.where(c == 0, m0, m1)
    base = jnp.where(c == 0, s * m0, NS * m0 + s * m1)
    pltpu.sync_copy(rowi_hbm.at[pl.ds(base, mmax)], idxr_v)
    pltpu.sync_copy(coli_hbm.at[pl.ds(base, mmax)], idxc_v)
    _zero_rows(rows_v.at[0], CHUNK, d)
    for t in range(grp):
        pltpu.sync_copy(rows_v.at[0],
                        acc_sh.at[pl.ds(s * rows_per_sub + t * CHUNK, CHUNK)])
    plsc.subcore_barrier()

    semg = sems[:NBUF]
    semsc = sems[NBUF:]
    for b in range(NB):
        pltpu.async_copy(ytab_sh.at[idxr_v.at[b]], rows_v.at[b], semg[b])

    @pl.loop(0, mmax, step=NBUF)
    def _chunks(g):
        @pl.when(g < m)
        def _():
            for bb in range(NBUF):
                j = g + bb
                b2 = (bb + NB) % NBUF

                pltpu.make_async_copy(
                    ytab_sh.at[idxr_v.at[j]], rows_v.at[bb], semg[bb]).wait()
                pltpu.async_copy(rows_v.at[bb], acc_sh.at[idxc_v.at[j]],
                                 semsc[bb], add=True)

                @pl.when(j >= NB)
                def _():
                    pltpu.make_async_copy(
                        rows_v.at[b2], acc_sh.at[idxc_v.at[j]],
                        semsc[b2]).wait()

                jn = j + NB

                @pl.when(jn < m)
                def _():
                    pltpu.async_copy(ytab_sh.at[idxr_v.at[jn]],
                                     rows_v.at[b2], semg[b2])

    for b in range(NB, NBUF):
        pltpu.make_async_copy(rows_v.at[b], acc_sh.at[idxc_v.at[0]],
                              semsc[b]).wait()

    plsc.subcore_barrier()
    for t in range(grp):
        off = s * rows_per_sub + t * CHUNK
        pltpu.sync_copy(acc_sh.at[pl.ds(off, CHUNK)],
                        out_hbm.at[c, pl.ds(off, CHUNK)])


def _hop_scratch(n_pad, d, mmax):
    return [
        pltpu.VMEM((mmax, CHUNK), jnp.int32),
        pltpu.VMEM((mmax, CHUNK), jnp.int32),
        pltpu.VMEM((NBUF, CHUNK, d), jnp.float32),
        pltpu.VMEM_SHARED((n_pad, d), jnp.float32),
        pltpu.VMEM_SHARED((n_pad, d), jnp.float32),
    ] + [pltpu.SemaphoreType.DMA] * (2 * NBUF)


_MESH = dict(core_axis_name="c", subcore_axis_name="s")


def _make_hop_first(n_pad, d, m0, m1):
    mmax = max(m0, m1)
    rows_per_sub = n_pad // NS

    @functools.partial(
        pl.kernel, mesh=plsc.VectorSubcoreMesh(**_MESH),
        out_type=jax.ShapeDtypeStruct((NC, n_pad, d), jnp.float32),
        compiler_params=pltpu.CompilerParams(use_tc_tiling_on_sc=False),
        scratch_types=_hop_scratch(n_pad, d, mmax))
    def hop(y_hbm, rowi_hbm, coli_hbm, out_hbm, idxr_v, idxc_v, rows_v,
            ytab_sh, acc_sh, *sems):
        c = lax.axis_index("c")
        s = lax.axis_index("s")
        off = s * rows_per_sub
        pltpu.sync_copy(y_hbm.at[pl.ds(off, rows_per_sub)],
                        ytab_sh.at[pl.ds(off, rows_per_sub)])
        _hop_edges(ytab_sh, acc_sh, rowi_hbm, coli_hbm, out_hbm, idxr_v,
                   idxc_v, rows_v, sems, c, s, m0, m1, mmax, n_pad, d)

    return hop


def _make_hop_mid(n_pad, d, m0, m1, lo_col):
    mmax = max(m0, m1)
    rows_per_sub = n_pad // NS
    grp = rows_per_sub // CHUNK

    @functools.partial(
        pl.kernel, mesh=plsc.VectorSubcoreMesh(**_MESH),
        out_type=jax.ShapeDtypeStruct((NC, n_pad, d), jnp.float32),
        compiler_params=pltpu.CompilerParams(use_tc_tiling_on_sc=False),
        scratch_types=[
            pltpu.VMEM((CHUNK, d), jnp.float32),
            pltpu.VMEM((CHUNK, d), jnp.float32),
            pltpu.VMEM((CHUNK, LANES), jnp.float32),
            pltpu.VMEM((CHUNK, d), jnp.float32),
        ] + _hop_scratch(n_pad, d, mmax))
    def hop(p_hbm, invd_hbm, rowi_hbm, coli_hbm, out_hbm, sa, sb, sd, yb,
            idxr_v, idxc_v, rows_v, ytab_sh, acc_sh, *sems):
        c = lax.axis_index("c")
        s = lax.axis_index("s")
        for t in range(grp):
            off = s * rows_per_sub + t * CHUNK
            pltpu.sync_copy(
                p_hbm.at[0, pl.ds(off, CHUNK), pl.ds(lo_col, d)], sa)
            pltpu.sync_copy(
                p_hbm.at[1, pl.ds(off, CHUNK), pl.ds(lo_col, d)], sb)
            pltpu.sync_copy(invd_hbm.at[pl.ds(off, CHUNK)], sd)

            def rowbody(i, _):
                for k in range(d // LANES):
                    sl = pl.ds(k * LANES, LANES)
                    yb[i, sl] = (sa[i, sl] + sb[i, sl]) * sd[i, :]
                return 0
            lax.fori_loop(0, CHUNK, rowbody, 0)
            pltpu.sync_copy(yb, ytab_sh.at[pl.ds(off, CHUNK)])
        _hop_edges(ytab_sh, acc_sh, rowi_hbm, coli_hbm, out_hbm, idxr_v,
                   idxc_v, rows_v, sems, c, s, m0, m1, mmax, n_pad, d)

    return hop


def _make_hop_relu(n_pad, d, m0, m1):
    mmax = max(m0, m1)
    rows_per_sub = n_pad // NS
    grp = rows_per_sub // CHUNK

    @functools.partial(
        pl.kernel, mesh=plsc.VectorSubcoreMesh(**_MESH),
        out_type=(jax.ShapeDtypeStruct((NC, n_pad, d), jnp.float32),
                  jax.ShapeDtypeStruct((n_pad, d), jnp.float32)),
        compiler_params=pltpu.CompilerParams(use_tc_tiling_on_sc=False),
        scratch_types=[
            pltpu.VMEM((CHUNK, d), jnp.float32),
            pltpu.VMEM((CHUNK, d), jnp.float32),
            pltpu.VMEM((CHUNK, d), jnp.float32),
            pltpu.VMEM((CHUNK, d), jnp.float32),
            pltpu.VMEM((CHUNK, d), jnp.float32),
            pltpu.VMEM((CHUNK, LANES), jnp.float32),
            pltpu.VMEM((CHUNK, d), jnp.float32),
            pltpu.VMEM((CHUNK, d), jnp.float32),
        ] + _hop_scratch(n_pad, d, mmax))
    def hop(pab_hbm, pc_hbm, xw_hbm, dinv_hbm, rowi_hbm, coli_hbm,
            out_hbm, h_hbm, sa, sb, sc0, sc1, sx, sd, yb, hb,
            idxr_v, idxc_v, rows_v, ytab_sh, acc_sh, *sems):
        c = lax.axis_index("c")
        s = lax.axis_index("s")
        for t in range(grp):
            off = s * rows_per_sub + t * CHUNK
            pltpu.sync_copy(pab_hbm.at[0, pl.ds(off, CHUNK), pl.ds(0, d)], sa)
            pltpu.sync_copy(pab_hbm.at[1, pl.ds(off, CHUNK), pl.ds(0, d)], sb)
            pltpu.sync_copy(pc_hbm.at[0, pl.ds(off, CHUNK)], sc0)
            pltpu.sync_copy(pc_hbm.at[1, pl.ds(off, CHUNK)], sc1)
            pltpu.sync_copy(xw_hbm.at[pl.ds(off, CHUNK)], sx)
            pltpu.sync_copy(dinv_hbm.at[pl.ds(off, CHUNK)], sd)

            def rowbody(i, _):
                for k in range(d // LANES):
                    sl = pl.ds(k * LANES, LANES)
                    z = sa[i, sl] + sb[i, sl] + sc0[i, sl] + sc1[i, sl]
                    h = jnp.maximum(sx[i, sl] + z * sd[i, :], 0.0)
                    hb[i, sl] = h
                    yb[i, sl] = h * sd[i, :]
                return 0
            lax.fori_loop(0, CHUNK, rowbody, 0)
            pltpu.sync_copy(hb, h_hbm.at[pl.ds(off, CHUNK)])
            pltpu.sync_copy(yb, ytab_sh.at[pl.ds(off, CHUNK)])
        _hop_edges(ytab_sh, acc_sh, rowi_hbm, coli_hbm, out_hbm, idxr_v,
                   idxc_v, rows_v, sems, c, s, m0, m1, mmax, n_pad, d)

    return hop


def _make_deg(n_pad, n_chunks):
    rows_per_sub = n_pad // NS
    grp = rows_per_sub // CHUNK

    @functools.partial(
        pl.kernel, mesh=plsc.VectorSubcoreMesh(**_MESH),
        out_type=jax.ShapeDtypeStruct((NC, n_pad, LANES), jnp.float32),
        compiler_params=pltpu.CompilerParams(use_tc_tiling_on_sc=False),
        scratch_types=[
            pltpu.VMEM((n_chunks, CHUNK), jnp.int32),
            pltpu.VMEM((CHUNK, LANES), jnp.float32),
            pltpu.VMEM_SHARED((n_pad, LANES), jnp.float32),
        ])
    def deg(coli_hbm, out_hbm, idxc_v, ones_v, acc_sh):
        c = lax.axis_index("c")
        s = lax.axis_index("s")
        wid = c * NS + s
        pltpu.sync_copy(coli_hbm.at[pl.ds(wid * n_chunks, n_chunks)], idxc_v)
        _zero_rows(ones_v, CHUNK, LANES)
        for t in range(grp):
            pltpu.sync_copy(
                ones_v, acc_sh.at[pl.ds(s * rows_per_sub + t * CHUNK, CHUNK)])
        def fill(i, _):
            ones_v[i, pl.ds(0, LANES)] = jnp.ones((LANES,), jnp.float32)
            return 0
        lax.fori_loop(0, CHUNK, fill, 0)
        plsc.subcore_barrier()

        def body(j, _):
            pltpu.sync_copy(ones_v, acc_sh.at[idxc_v.at[j]], add=True)
            return 0
        lax.fori_loop(0, n_chunks, body, 0)
        plsc.subcore_barrier()
        for t in range(grp):
            off = s * rows_per_sub + t * CHUNK
            pltpu.sync_copy(acc_sh.at[pl.ds(off, CHUNK)],
                            out_hbm.at[c, pl.ds(off, CHUNK)])

    return deg


_BLK = 1024


def _proj_body(degp_ref, x_ref, w_ref, b_ref, xw_ref, y12_ref, dinv_ref,
               invd_ref):
    dsum = degp_ref[0] + degp_ref[1]
    pos = dsum > 0
    dinv = jnp.where(pos, lax.rsqrt(dsum), 0.0)
    dinv_ref[...] = dinv
    invd_ref[...] = jnp.where(pos, 1.0 / dsum, 0.0)
    d1 = dinv[:, 0:1]
    xb = x_ref[...]
    dd = xb.shape[1]
    hdim = xw_ref.shape[1]
    xw_ref[...] = (
        jnp.dot(xb, w_ref[0:dd], preferred_element_type=jnp.float32)
        + b_ref[...])
    u1 = jnp.dot(xb, w_ref[dd:2 * dd], preferred_element_type=jnp.float32)
    u2 = jnp.dot(xb, w_ref[2 * dd:3 * dd], preferred_element_type=jnp.float32)
    y12_ref[:, 0:hdim] = u1 * d1
    y12_ref[:, hdim:2 * hdim] = u2 * d1


def _out_body(h_ref, qa_ref, qb_ref, dinv_ref, w_ref, b_ref, out_ref):
    d1 = dinv_ref[...][:, 0:1]
    x1 = (qa_ref[0] + qa_ref[1]) * d1
    x2 = (qb_ref[0] + qb_ref[1]) * d1
    hh = h_ref.shape[1]
    acc = jnp.dot(h_ref[...], w_ref[0:hh], preferred_element_type=jnp.float32)
    acc += jnp.dot(x1, w_ref[hh:2 * hh], preferred_element_type=jnp.float32)
    acc += jnp.dot(x2, w_ref[2 * hh:3 * hh], preferred_element_type=jnp.float32)
    out_ref[...] = acc + b_ref[...]


def _row_spec(d):
    return pl.BlockSpec((_BLK, d), lambda i: (i, 0))


def _pair_spec(d):
    return pl.BlockSpec((NC, _BLK, d), lambda i: (0, i, 0))


def _full_spec(shape):
    return pl.BlockSpec(shape, lambda i: tuple(0 for _ in shape))


def kernel(x, edge_index, W1, b1, W2, b2):
    n, dd = x.shape
    hdim = W1.shape[1]
    e = edge_index.shape[1]

    n_pad = -(-n // (NS * CHUNK)) * (NS * CHUNK)
    e_pad = -(-e // (NW * CHUNK * NB)) * (NW * CHUNK * NB)
    n_chunks = e_pad // (NW * CHUNK)
    mm = 2 * n_chunks
    m0_32, m1_32 = 80, 80
    m0_16, m1_16 = 80, 80
    assert m0_32 + m1_32 == mm and m0_16 + m1_16 == mm
    padc = max(m0_32, m0_16, m1_32, m1_16)
    grid = n_pad // _BLK

    row = jnp.pad(edge_index[0], (0, e_pad - e))
    col = jnp.pad(edge_index[1], (0, e_pad - e),
                  constant_values=n)
    rowi = jnp.pad(row.reshape(NW * n_chunks, CHUNK), ((0, padc), (0, 0)))
    coli = jnp.pad(col.reshape(NW * n_chunks, CHUNK), ((0, padc), (0, 0)),
                   constant_values=n)
    x_pad = jnp.pad(x, ((0, n_pad - n), (0, 0)))

    hopA = _make_hop_first(n_pad, 2 * hdim, m0_32, m1_32)
    hopC = _make_hop_mid(n_pad, hdim, m0_16, m1_16, hdim)
    hopD = _make_hop_relu(n_pad, hdim, m0_16, m1_16)
    hopE = _make_hop_mid(n_pad, hdim, m0_16, m1_16, 0)
    degk = _make_deg(n_pad, n_chunks)

    degp = degk(coli)

    xw, y12, dinv, invd = pl.pallas_call(
        _proj_body,
        grid=(grid,),
        in_specs=[_pair_spec(LANES), _row_spec(dd), _full_spec(W1.shape),
                  _full_spec((1, hdim))],
        out_specs=[_row_spec(hdim), _row_spec(2 * hdim), _row_spec(LANES),
                   _row_spec(LANES)],
        out_shape=[jax.ShapeDtypeStruct((n_pad, hdim), jnp.float32),
                   jax.ShapeDtypeStruct((n_pad, 2 * hdim), jnp.float32),
                   jax.ShapeDtypeStruct((n_pad, LANES), jnp.float32),
                   jax.ShapeDtypeStruct((n_pad, LANES), jnp.float32)],
    )(degp, x_pad, W1, b1.reshape(1, hdim))

    Pab = hopA(y12, rowi, coli)
    Pc = hopC(Pab, invd, rowi, coli)
    Q1, h = hopD(Pab, Pc, xw, dinv, rowi, coli)
    Q2 = hopE(Q1, invd, rowi, coli)

    out = pl.pallas_call(
        _out_body,
        grid=(grid,),
        in_specs=[_row_spec(hdim), _pair_spec(hdim), _pair_spec(hdim),
                  _row_spec(LANES), _full_spec(W2.shape), _full_spec((1, dd))],
        out_specs=_row_spec(dd),
        out_shape=jax.ShapeDtypeStruct((n_pad, dd), jnp.float32),
    )(h, Q1, Q2, dinv, W2, b2.reshape(1, dd))

    return out[:n]


# speedup vs baseline: 1.0419x; 1.0172x over previous
import functools

import jax
import jax.numpy as jnp
from jax import lax
from jax.experimental import pallas as pl
from jax.experimental.pallas import tpu as pltpu
from jax.experimental.pallas import tpu_sc as plsc

NC = 2
NS = 16
LANES = 16
NW = NC * NS
CHUNK = 128
NB = 4
NBUF = 2 * NB


def _zero_rows(buf, nrows, d):
    def body(i, _):
        for k in range(d // LANES):
            buf[i, pl.ds(k * LANES, LANES)] = jnp.zeros((LANES,), jnp.float32)
        return 0
    lax.fori_loop(0, nrows, body, 0)


def _hop_edges(ytab_sh, acc_sh, rowi_hbm, coli_hbm, out_hbm, idxr_v, idxc_v,
               rows_v, sems, c, s, m0, m1, mmax, n_pad, d):
    rows_per_sub = n_pad // NS
    grp = rows_per_sub // CHUNK
    m = jnp.where(c == 0, m0, m1)
    base = jnp.where(c == 0, s * m0, NS * m0 + s * m1)
    pltpu.sync_copy(rowi_hbm.at[pl.ds(base, mmax)], idxr_v)
    pltpu.sync_copy(coli_hbm.at[pl.ds(base, mmax)], idxc_v)
    _zero_rows(rows_v.at[0], CHUNK, d)
    for t in range(grp):
        pltpu.sync_copy(rows_v.at[0],
                        acc_sh.at[pl.ds(s * rows_per_sub + t * CHUNK, CHUNK)])
    plsc.subcore_barrier()

    semg = sems[:NBUF]
    semsc = sems[NBUF:]
    for b in range(NB):
        pltpu.async_copy(ytab_sh.at[idxr_v.at[b]], rows_v.at[b], semg[b])

    @pl.loop(0, mmax, step=NBUF)
    def _chunks(g):
        @pl.when(g < m)
        def _():
            for bb in range(NBUF):
                j = g + bb
                b2 = (bb + NB) % NBUF

                pltpu.make_async_copy(
                    ytab_sh.at[idxr_v.at[j]], rows_v.at[bb], semg[bb]).wait()
                pltpu.async_copy(rows_v.at[bb], acc_sh.at[idxc_v.at[j]],
                                 semsc[bb], add=True)

                @pl.when(j >= NB)
                def _():
                    pltpu.make_async_copy(
                        rows_v.at[b2], acc_sh.at[idxc_v.at[j]],
                        semsc[b2]).wait()

                jn = j + NB

                @pl.when(jn < m)
                def _():
                    pltpu.async_copy(ytab_sh.at[idxr_v.at[jn]],
                                     rows_v.at[b2], semg[b2])

    for b in range(NB, NBUF):
        pltpu.make_async_copy(rows_v.at[b], acc_sh.at[idxc_v.at[0]],
                              semsc[b]).wait()

    plsc.subcore_barrier()
    for t in range(grp):
        off = s * rows_per_sub + t * CHUNK
        pltpu.sync_copy(acc_sh.at[pl.ds(off, CHUNK)],
                        out_hbm.at[c, pl.ds(off, CHUNK)])


def _hop_scratch(n_pad, d, mmax):
    return [
        pltpu.VMEM((mmax, CHUNK), jnp.int32),
        pltpu.VMEM((mmax, CHUNK), jnp.int32),
        pltpu.VMEM((NBUF, CHUNK, d), jnp.float32),
        pltpu.VMEM_SHARED((n_pad, d), jnp.float32),
        pltpu.VMEM_SHARED((n_pad, d), jnp.float32),
    ] + [pltpu.SemaphoreType.DMA] * (2 * NBUF)


_MESH = dict(core_axis_name="c", subcore_axis_name="s")


def _make_hop_first(n_pad, d, m0, m1):
    mmax = max(m0, m1)
    rows_per_sub = n_pad // NS

    @functools.partial(
        pl.kernel, mesh=plsc.VectorSubcoreMesh(**_MESH),
        out_type=jax.ShapeDtypeStruct((NC, n_pad, d), jnp.float32),
        compiler_params=pltpu.CompilerParams(use_tc_tiling_on_sc=False),
        scratch_types=_hop_scratch(n_pad, d, mmax))
    def hop(y_hbm, rowi_hbm, coli_hbm, out_hbm, idxr_v, idxc_v, rows_v,
            ytab_sh, acc_sh, *sems):
        c = lax.axis_index("c")
        s = lax.axis_index("s")
        off = s * rows_per_sub
        pltpu.sync_copy(y_hbm.at[pl.ds(off, rows_per_sub), pl.ds(0, d)],
                        ytab_sh.at[pl.ds(off, rows_per_sub)])
        _hop_edges(ytab_sh, acc_sh, rowi_hbm, coli_hbm, out_hbm, idxr_v,
                   idxc_v, rows_v, sems, c, s, m0, m1, mmax, n_pad, d)

    return hop


def _make_hop_mid(n_pad, d, m0, m1, lo_col, invd_col):
    mmax = max(m0, m1)
    rows_per_sub = n_pad // NS
    grp = rows_per_sub // CHUNK

    @functools.partial(
        pl.kernel, mesh=plsc.VectorSubcoreMesh(**_MESH),
        out_type=jax.ShapeDtypeStruct((NC, n_pad, d), jnp.float32),
        compiler_params=pltpu.CompilerParams(use_tc_tiling_on_sc=False),
        scratch_types=[
            pltpu.VMEM((CHUNK, d), jnp.float32),
            pltpu.VMEM((CHUNK, d), jnp.float32),
            pltpu.VMEM((CHUNK, LANES), jnp.float32),
            pltpu.VMEM((CHUNK, d), jnp.float32),
        ] + _hop_scratch(n_pad, d, mmax))
    def hop(p_hbm, invd_hbm, rowi_hbm, coli_hbm, out_hbm, sa, sb, sd, yb,
            idxr_v, idxc_v, rows_v, ytab_sh, acc_sh, *sems):
        c = lax.axis_index("c")
        s = lax.axis_index("s")
        for t in range(grp):
            off = s * rows_per_sub + t * CHUNK
            pltpu.sync_copy(
                p_hbm.at[0, pl.ds(off, CHUNK), pl.ds(lo_col, d)], sa)
            pltpu.sync_copy(
                p_hbm.at[1, pl.ds(off, CHUNK), pl.ds(lo_col, d)], sb)
            pltpu.sync_copy(
                invd_hbm.at[pl.ds(off, CHUNK), pl.ds(invd_col, LANES)], sd)

            def rowbody(i, _):
                for k in range(d // LANES):
                    sl = pl.ds(k * LANES, LANES)
                    yb[i, sl] = (sa[i, sl] + sb[i, sl]) * sd[i, :]
                return 0
            lax.fori_loop(0, CHUNK, rowbody, 0)
            pltpu.sync_copy(yb, ytab_sh.at[pl.ds(off, CHUNK)])
        _hop_edges(ytab_sh, acc_sh, rowi_hbm, coli_hbm, out_hbm, idxr_v,
                   idxc_v, rows_v, sems, c, s, m0, m1, mmax, n_pad, d)

    return hop


def _make_hop_relu(n_pad, d, m0, m1):
    mmax = max(m0, m1)
    rows_per_sub = n_pad // NS
    grp = rows_per_sub // CHUNK

    @functools.partial(
        pl.kernel, mesh=plsc.VectorSubcoreMesh(**_MESH),
        out_type=(jax.ShapeDtypeStruct((NC, n_pad, d), jnp.float32),
                  jax.ShapeDtypeStruct((n_pad, 128), jnp.float32)),
        compiler_params=pltpu.CompilerParams(use_tc_tiling_on_sc=False),
        scratch_types=[
            pltpu.VMEM((CHUNK, d), jnp.float32),
            pltpu.VMEM((CHUNK, d), jnp.float32),
            pltpu.VMEM((CHUNK, d), jnp.float32),
            pltpu.VMEM((CHUNK, d), jnp.float32),
            pltpu.VMEM((CHUNK, d), jnp.float32),
            pltpu.VMEM((CHUNK, LANES), jnp.float32),
            pltpu.VMEM((CHUNK, d), jnp.float32),
            pltpu.VMEM((CHUNK, d), jnp.float32),
        ] + _hop_scratch(n_pad, d, mmax))
    def hop(pab_hbm, pc_hbm, pack_hbm, rowi_hbm, coli_hbm,
            out_hbm, h_hbm, sa, sb, sc0, sc1, sx, sd, yb, hb,
            idxr_v, idxc_v, rows_v, ytab_sh, acc_sh, *sems):
        c = lax.axis_index("c")
        s = lax.axis_index("s")
        for t in range(grp):
            off = s * rows_per_sub + t * CHUNK
            pltpu.sync_copy(pab_hbm.at[0, pl.ds(off, CHUNK), pl.ds(0, d)], sa)
            pltpu.sync_copy(pab_hbm.at[1, pl.ds(off, CHUNK), pl.ds(0, d)], sb)
            pltpu.sync_copy(pc_hbm.at[0, pl.ds(off, CHUNK)], sc0)
            pltpu.sync_copy(pc_hbm.at[1, pl.ds(off, CHUNK)], sc1)
            pltpu.sync_copy(
                pack_hbm.at[pl.ds(off, CHUNK), pl.ds(2 * d, d)], sx)
            pltpu.sync_copy(
                pack_hbm.at[pl.ds(off, CHUNK), pl.ds(3 * d, LANES)], sd)

            def rowbody(i, _):
                for k in range(d // LANES):
                    sl = pl.ds(k * LANES, LANES)
                    z = sa[i, sl] + sb[i, sl] + sc0[i, sl] + sc1[i, sl]
                    h = jnp.maximum(sx[i, sl] + z * sd[i, :], 0.0)
                    hb[i, sl] = h
                    yb[i, sl] = h * sd[i, :]
                return 0
            lax.fori_loop(0, CHUNK, rowbody, 0)
            pltpu.sync_copy(hb, h_hbm.at[pl.ds(off, CHUNK),
                                         pl.ds(0, d)])
            pltpu.sync_copy(yb, ytab_sh.at[pl.ds(off, CHUNK)])
        _hop_edges(ytab_sh, acc_sh, rowi_hbm, coli_hbm, out_hbm, idxr_v,
                   idxc_v, rows_v, sems, c, s, m0, m1, mmax, n_pad, d)

    return hop


def _make_deg(n_pad, n_chunks):
    rows_per_sub = n_pad // NS
    grp = rows_per_sub // CHUNK

    @functools.partial(
        pl.kernel, mesh=plsc.VectorSubcoreMesh(**_MESH),
        out_type=jax.ShapeDtypeStruct((NC, n_pad, LANES), jnp.float32),
        compiler_params=pltpu.CompilerParams(use_tc_tiling_on_sc=False),
        scratch_types=[
            pltpu.VMEM((n_chunks, CHUNK), jnp.int32),
            pltpu.VMEM((CHUNK, LANES), jnp.float32),
            pltpu.VMEM_SHARED((n_pad, LANES), jnp.float32),
        ])
    def deg(coli_hbm, out_hbm, idxc_v, ones_v, acc_sh):
        c = lax.axis_index("c")
        s = lax.axis_index("s")
        wid = c * NS + s
        pltpu.sync_copy(coli_hbm.at[pl.ds(wid * n_chunks, n_chunks)], idxc_v)
        _zero_rows(ones_v, CHUNK, LANES)
        for t in range(grp):
            pltpu.sync_copy(
                ones_v, acc_sh.at[pl.ds(s * rows_per_sub + t * CHUNK, CHUNK)])
        def fill(i, _):
            ones_v[i, pl.ds(0, LANES)] = jnp.ones((LANES,), jnp.float32)
            return 0
        lax.fori_loop(0, CHUNK, fill, 0)
        plsc.subcore_barrier()

        def body(j, _):
            pltpu.sync_copy(ones_v, acc_sh.at[idxc_v.at[j]], add=True)
            return 0
        lax.fori_loop(0, n_chunks, body, 0)
        plsc.subcore_barrier()
        for t in range(grp):
            off = s * rows_per_sub + t * CHUNK
            pltpu.sync_copy(acc_sh.at[pl.ds(off, CHUNK)],
                            out_hbm.at[c, pl.ds(off, CHUNK)])

    return deg


_BLK = 1024


def _proj_body(degp_ref, x_ref, w_ref, b_ref, pack_ref):
    dsum = degp_ref[0] + degp_ref[1]
    pos = dsum > 0
    dinv = jnp.where(pos, lax.rsqrt(dsum), 0.0)
    hdim = w_ref.shape[1] // 3
    r = jnp.dot(x_ref[...], w_ref[...], preferred_element_type=jnp.float32)
    pack_ref[:, 0:2 * hdim] = r[:, hdim:3 * hdim] * dinv[:, 0:1]
    pack_ref[:, 2 * hdim:3 * hdim] = r[:, 0:hdim] + b_ref[...]
    pack_ref[:, 3 * hdim:3 * hdim + LANES] = dinv
    pack_ref[:, 3 * hdim + LANES:3 * hdim + 2 * LANES] = (
        jnp.where(pos, 1.0 / dsum, 0.0))
    pack_ref[:, 3 * hdim + 2 * LANES:] = jnp.zeros(
        (pack_ref.shape[0], pack_ref.shape[1] - 3 * hdim - 2 * LANES),
        jnp.float32)


def _out_body(h_ref, qa_ref, qb_ref, pack_ref, w_ref, b_ref, out_ref):
    hh = w_ref.shape[0] // 3
    d1 = pack_ref[:, 3 * hh:3 * hh + 1]
    x1 = (qa_ref[0] + qa_ref[1]) * d1
    x2 = (qb_ref[0] + qb_ref[1]) * d1
    hx = jnp.concatenate([h_ref[:, 0:hh], x1, x2], axis=1)
    out_ref[...] = (
        jnp.dot(hx, w_ref[...], preferred_element_type=jnp.float32)
        + b_ref[...])


def _row_spec(d):
    return pl.BlockSpec((_BLK, d), lambda i: (i, 0))


def _pair_spec(d):
    return pl.BlockSpec((NC, _BLK, d), lambda i: (0, i, 0))


def _full_spec(shape):
    return pl.BlockSpec(shape, lambda i: tuple(0 for _ in shape))


def kernel(x, edge_index, W1, b1, W2, b2):
    n, dd = x.shape
    hdim = W1.shape[1]
    e = edge_index.shape[1]

    n_pad = -(-n // (NS * CHUNK)) * (NS * CHUNK)
    e_pad = -(-e // (NW * CHUNK * NB)) * (NW * CHUNK * NB)
    n_chunks = e_pad // (NW * CHUNK)
    mm = 2 * n_chunks
    m0_32, m1_32 = 80, 80
    m0_16, m1_16 = 80, 80
    assert m0_32 + m1_32 == mm and m0_16 + m1_16 == mm
    padc = max(m0_32, m0_16, m1_32, m1_16)
    grid = n_pad // _BLK

    row = jnp.pad(edge_index[0], (0, e_pad - e))
    col = jnp.pad(edge_index[1], (0, e_pad - e),
                  constant_values=n)
    rowi = jnp.pad(row.reshape(NW * n_chunks, CHUNK), ((0, padc), (0, 0)))
    coli = jnp.pad(col.reshape(NW * n_chunks, CHUNK), ((0, padc), (0, 0)),
                   constant_values=n)
    x_pad = jnp.pad(x, ((0, n_pad - n), (0, 0)))

    invd_col = 3 * hdim + LANES

    hopA = _make_hop_first(n_pad, 2 * hdim, m0_32, m1_32)
    hopC = _make_hop_mid(n_pad, hdim, m0_16, m1_16, hdim, invd_col)
    hopD = _make_hop_relu(n_pad, hdim, m0_16, m1_16)
    hopE = _make_hop_mid(n_pad, hdim, m0_16, m1_16, 0, invd_col)
    degk = _make_deg(n_pad, n_chunks)

    degp = degk(coli)

    w1cat = jnp.concatenate(
        [W1[0:dd], W1[dd:2 * dd], W1[2 * dd:3 * dd]], axis=1)

    pack = pl.pallas_call(
        _proj_body,
        grid=(grid,),
        in_specs=[_pair_spec(LANES), _row_spec(dd), _full_spec(w1cat.shape),
                  _full_spec((1, hdim))],
        out_specs=_row_spec(128),
        out_shape=jax.ShapeDtypeStruct((n_pad, 128), jnp.float32),
    )(degp, x_pad, w1cat, b1.reshape(1, hdim))

    Pab = hopA(pack, rowi, coli)
    Pc = hopC(Pab, pack, rowi, coli)
    Q1, h = hopD(Pab, Pc, pack, rowi, coli)
    Q2 = hopE(Q1, pack, rowi, coli)

    out = pl.pallas_call(
        _out_body,
        grid=(grid,),
        in_specs=[_row_spec(128), _pair_spec(hdim), _pair_spec(hdim),
                  _row_spec(128), _full_spec(W2.shape), _full_spec((1, dd))],
        out_specs=_row_spec(dd),
        out_shape=jax.ShapeDtypeStruct((n_pad, dd), jnp.float32),
    )(h, Q1, Q2, pack, W2, b2.reshape(1, dd))

    return out[:n]


# speedup vs baseline: 1.0564x; 1.0139x over previous
import functools

import jax
import jax.numpy as jnp
from jax import lax
from jax.experimental import pallas as pl
from jax.experimental.pallas import tpu as pltpu
from jax.experimental.pallas import tpu_sc as plsc

NC = 2
NS = 16
LANES = 16
NW = NC * NS
CHUNK = 128
NB = 4
NBUF = 2 * NB


def _zero_rows(buf, nrows, d):
    def body(i, _):
        for k in range(d // LANES):
            buf[i, pl.ds(k * LANES, LANES)] = jnp.zeros((LANES,), jnp.float32)
        return 0
    lax.fori_loop(0, nrows, body, 0)


def _hop_edges(ytab_sh, acc_sh, rowi_hbm, coli_hbm, out_hbm, idxr_v, idxc_v,
               rows_v, sems, c, s, m0, m1, mmax, n_pad, d):
    rows_per_sub = n_pad // NS
    grp = rows_per_sub // CHUNK
    m = jnp.where(c == 0, m0, m1)
    base = jnp.where(c == 0, s * m0, NS * m0 + s * m1)
    pltpu.sync_copy(rowi_hbm.at[pl.ds(base, mmax)], idxr_v)
    pltpu.sync_copy(coli_hbm.at[pl.ds(base, mmax)], idxc_v)
    _zero_rows(rows_v.at[0], CHUNK, d)
    for t in range(grp):
        pltpu.sync_copy(rows_v.at[0],
                        acc_sh.at[pl.ds(s * rows_per_sub + t * CHUNK, CHUNK)])
    plsc.subcore_barrier()

    semg = sems[:NBUF]
    semsc = sems[NBUF:]
    for b in range(NB):
        pltpu.async_copy(ytab_sh.at[idxr_v.at[b]], rows_v.at[b], semg[b])

    @pl.loop(0, mmax, step=NBUF)
    def _chunks(g):
        @pl.when(g < m)
        def _():
            for bb in range(NBUF):
                j = g + bb
                b2 = (bb + NB) % NBUF

                pltpu.make_async_copy(
                    ytab_sh.at[idxr_v.at[j]], rows_v.at[bb], semg[bb]).wait()
                pltpu.async_copy(rows_v.at[bb], acc_sh.at[idxc_v.at[j]],
                                 semsc[bb], add=True)

                @pl.when(j >= NB)
                def _():
                    pltpu.make_async_copy(
                        rows_v.at[b2], acc_sh.at[idxc_v.at[j]],
                        semsc[b2]).wait()

                jn = j + NB

                @pl.when(jn < m)
                def _():
                    pltpu.async_copy(ytab_sh.at[idxr_v.at[jn]],
                                     rows_v.at[b2], semg[b2])

    for b in range(NB, NBUF):
        pltpu.make_async_copy(rows_v.at[b], acc_sh.at[idxc_v.at[0]],
                              semsc[b]).wait()

    plsc.subcore_barrier()
    for t in range(grp):
        off = s * rows_per_sub + t * CHUNK
        pltpu.sync_copy(acc_sh.at[pl.ds(off, CHUNK)],
                        out_hbm.at[c, pl.ds(off, CHUNK)])


def _hop_scratch(n_pad, d, mmax):
    return [
        pltpu.VMEM((mmax, CHUNK), jnp.int32),
        pltpu.VMEM((mmax, CHUNK), jnp.int32),
        pltpu.VMEM((NBUF, CHUNK, d), jnp.float32),
        pltpu.VMEM_SHARED((n_pad, d), jnp.float32),
        pltpu.VMEM_SHARED((n_pad, d), jnp.float32),
    ] + [pltpu.SemaphoreType.DMA] * (2 * NBUF)


_MESH = dict(core_axis_name="c", subcore_axis_name="s")


def _make_hop_first(n_pad, d, m0, m1):
    mmax = max(m0, m1)
    rows_per_sub = n_pad // NS

    @functools.partial(
        pl.kernel, mesh=plsc.VectorSubcoreMesh(**_MESH),
        out_type=jax.ShapeDtypeStruct((NC, n_pad, d), jnp.float32),
        compiler_params=pltpu.CompilerParams(use_tc_tiling_on_sc=False),
        scratch_types=_hop_scratch(n_pad, d, mmax))
    def hop(y_hbm, rowi_hbm, coli_hbm, out_hbm, idxr_v, idxc_v, rows_v,
            ytab_sh, acc_sh, *sems):
        c = lax.axis_index("c")
        s = lax.axis_index("s")
        off = s * rows_per_sub
        pltpu.sync_copy(y_hbm.at[pl.ds(off, rows_per_sub), pl.ds(0, d)],
                        ytab_sh.at[pl.ds(off, rows_per_sub)])
        _hop_edges(ytab_sh, acc_sh, rowi_hbm, coli_hbm, out_hbm, idxr_v,
                   idxc_v, rows_v, sems, c, s, m0, m1, mmax, n_pad, d)

    return hop


def _make_hop_mid(n_pad, d, m0, m1, lo_col, invd_col):
    mmax = max(m0, m1)
    rows_per_sub = n_pad // NS
    grp = rows_per_sub // CHUNK

    @functools.partial(
        pl.kernel, mesh=plsc.VectorSubcoreMesh(**_MESH),
        out_type=jax.ShapeDtypeStruct((NC, n_pad, d), jnp.float32),
        compiler_params=pltpu.CompilerParams(use_tc_tiling_on_sc=False),
        scratch_types=[
            pltpu.VMEM((CHUNK, d), jnp.float32),
            pltpu.VMEM((CHUNK, d), jnp.float32),
            pltpu.VMEM((CHUNK, LANES), jnp.float32),
            pltpu.VMEM((CHUNK, d), jnp.float32),
        ] + _hop_scratch(n_pad, d, mmax))
    def hop(p_hbm, invd_hbm, rowi_hbm, coli_hbm, out_hbm, sa, sb, sd, yb,
            idxr_v, idxc_v, rows_v, ytab_sh, acc_sh, *sems):
        c = lax.axis_index("c")
        s = lax.axis_index("s")
        for t in range(grp):
            off = s * rows_per_sub + t * CHUNK
            pltpu.sync_copy(
                p_hbm.at[0, pl.ds(off, CHUNK), pl.ds(lo_col, d)], sa)
            pltpu.sync_copy(
                p_hbm.at[1, pl.ds(off, CHUNK), pl.ds(lo_col, d)], sb)
            pltpu.sync_copy(
                invd_hbm.at[pl.ds(off, CHUNK), pl.ds(invd_col, LANES)], sd)

            def rowbody(i, _):
                for k in range(d // LANES):
                    sl = pl.ds(k * LANES, LANES)
                    yb[i, sl] = (sa[i, sl] + sb[i, sl]) * sd[i, :]
                return 0
            lax.fori_loop(0, CHUNK, rowbody, 0)
            pltpu.sync_copy(yb, ytab_sh.at[pl.ds(off, CHUNK)])
        _hop_edges(ytab_sh, acc_sh, rowi_hbm, coli_hbm, out_hbm, idxr_v,
                   idxc_v, rows_v, sems, c, s, m0, m1, mmax, n_pad, d)

    return hop


def _make_hop_relu(n_pad, d, m0, m1):
    mmax = max(m0, m1)
    rows_per_sub = n_pad // NS
    grp = rows_per_sub // CHUNK

    @functools.partial(
        pl.kernel, mesh=plsc.VectorSubcoreMesh(**_MESH),
        out_type=(jax.ShapeDtypeStruct((NC, n_pad, d), jnp.float32),
                  jax.ShapeDtypeStruct((n_pad, 128), jnp.float32)),
        compiler_params=pltpu.CompilerParams(use_tc_tiling_on_sc=False),
        scratch_types=[
            pltpu.VMEM((CHUNK, d), jnp.float32),
            pltpu.VMEM((CHUNK, d), jnp.float32),
            pltpu.VMEM((CHUNK, d), jnp.float32),
            pltpu.VMEM((CHUNK, d), jnp.float32),
            pltpu.VMEM((CHUNK, d), jnp.float32),
            pltpu.VMEM((CHUNK, LANES), jnp.float32),
            pltpu.VMEM((CHUNK, d), jnp.float32),
            pltpu.VMEM((CHUNK, d), jnp.float32),
        ] + _hop_scratch(n_pad, d, mmax))
    def hop(pab_hbm, pc_hbm, pack_hbm, rowi_hbm, coli_hbm,
            out_hbm, h_hbm, sa, sb, sc0, sc1, sx, sd, yb, hb,
            idxr_v, idxc_v, rows_v, ytab_sh, acc_sh, *sems):
        c = lax.axis_index("c")
        s = lax.axis_index("s")
        for t in range(grp):
            off = s * rows_per_sub + t * CHUNK
            pltpu.sync_copy(pab_hbm.at[0, pl.ds(off, CHUNK), pl.ds(0, d)], sa)
            pltpu.sync_copy(pab_hbm.at[1, pl.ds(off, CHUNK), pl.ds(0, d)], sb)
            pltpu.sync_copy(pc_hbm.at[0, pl.ds(off, CHUNK)], sc0)
            pltpu.sync_copy(pc_hbm.at[1, pl.ds(off, CHUNK)], sc1)
            pltpu.sync_copy(
                pack_hbm.at[pl.ds(off, CHUNK), pl.ds(2 * d, d)], sx)
            pltpu.sync_copy(
                pack_hbm.at[pl.ds(off, CHUNK), pl.ds(3 * d, LANES)], sd)

            def rowbody(i, _):
                for k in range(d // LANES):
                    sl = pl.ds(k * LANES, LANES)
                    z = sa[i, sl] + sb[i, sl] + sc0[i, sl] + sc1[i, sl]
                    h = jnp.maximum(sx[i, sl] + z * sd[i, :], 0.0)
                    hb[i, sl] = h
                    yb[i, sl] = h * sd[i, :]
                return 0
            lax.fori_loop(0, CHUNK, rowbody, 0)
            pltpu.sync_copy(hb, h_hbm.at[pl.ds(off, CHUNK),
                                         pl.ds(0, d)])
            pltpu.sync_copy(yb, ytab_sh.at[pl.ds(off, CHUNK)])
        _hop_edges(ytab_sh, acc_sh, rowi_hbm, coli_hbm, out_hbm, idxr_v,
                   idxc_v, rows_v, sems, c, s, m0, m1, mmax, n_pad, d)

    return hop


def _make_deg(n_pad, n_chunks):
    rows_per_sub = n_pad // NS
    grp = rows_per_sub // CHUNK

    @functools.partial(
        pl.kernel, mesh=plsc.VectorSubcoreMesh(**_MESH),
        out_type=jax.ShapeDtypeStruct((NC, n_pad, LANES), jnp.float32),
        compiler_params=pltpu.CompilerParams(use_tc_tiling_on_sc=False),
        scratch_types=[
            pltpu.VMEM((n_chunks, CHUNK), jnp.int32),
            pltpu.VMEM((CHUNK, LANES), jnp.float32),
            pltpu.VMEM_SHARED((n_pad, LANES), jnp.float32),
        ])
    def deg(coli_hbm, out_hbm, idxc_v, ones_v, acc_sh):
        c = lax.axis_index("c")
        s = lax.axis_index("s")
        wid = c * NS + s
        pltpu.sync_copy(coli_hbm.at[pl.ds(wid * n_chunks, n_chunks)], idxc_v)
        _zero_rows(ones_v, CHUNK, LANES)
        for t in range(grp):
            pltpu.sync_copy(
                ones_v, acc_sh.at[pl.ds(s * rows_per_sub + t * CHUNK, CHUNK)])
        def fill(i, _):
            ones_v[i, pl.ds(0, LANES)] = jnp.ones((LANES,), jnp.float32)
            return 0
        lax.fori_loop(0, CHUNK, fill, 0)
        plsc.subcore_barrier()

        def body(j, _):
            pltpu.sync_copy(ones_v, acc_sh.at[idxc_v.at[j]], add=True)
            return 0
        lax.fori_loop(0, n_chunks, body, 0)
        plsc.subcore_barrier()
        for t in range(grp):
            off = s * rows_per_sub + t * CHUNK
            pltpu.sync_copy(acc_sh.at[pl.ds(off, CHUNK)],
                            out_hbm.at[c, pl.ds(off, CHUNK)])

    return deg


_BLK = 1024


def _proj_body(degp_ref, x_ref, w_ref, b_ref, pack_ref):
    dsum = degp_ref[0] + degp_ref[1]
    pos = dsum > 0
    dinv = jnp.where(pos, lax.rsqrt(dsum), 0.0)
    hdim = w_ref.shape[1] // 3
    r = jnp.dot(x_ref[...], w_ref[...], preferred_element_type=jnp.float32)
    pack_ref[:, 0:2 * hdim] = r[:, hdim:3 * hdim] * dinv[:, 0:1]
    pack_ref[:, 2 * hdim:3 * hdim] = r[:, 0:hdim] + b_ref[...]
    pack_ref[:, 3 * hdim:3 * hdim + LANES] = dinv
    pack_ref[:, 3 * hdim + LANES:3 * hdim + 2 * LANES] = (
        jnp.where(pos, 1.0 / dsum, 0.0))
    pack_ref[:, 3 * hdim + 2 * LANES:] = jnp.zeros(
        (pack_ref.shape[0], pack_ref.shape[1] - 3 * hdim - 2 * LANES),
        jnp.float32)


def _out_body(h_ref, qa_ref, qb_ref, pack_ref, w_ref, b_ref, out_ref):
    hh = w_ref.shape[0] // 3
    d1 = pack_ref[:, 3 * hh:3 * hh + 1]
    x1 = (qa_ref[0] + qa_ref[1]) * d1
    x2 = (qb_ref[0] + qb_ref[1]) * d1
    hx = jnp.concatenate([h_ref[:, 0:hh], x1, x2], axis=1)
    out_ref[...] = (
        jnp.dot(hx, w_ref[...], preferred_element_type=jnp.float32)
        + b_ref[...])


def _row_spec(d):
    return pl.BlockSpec((_BLK, d), lambda i: (i, 0))


def _pair_spec(d):
    return pl.BlockSpec((NC, _BLK, d), lambda i: (0, i, 0))


def _full_spec(shape):
    return pl.BlockSpec(shape, lambda i: tuple(0 for _ in shape))


def kernel(x, edge_index, W1, b1, W2, b2):
    n, dd = x.shape
    hdim = W1.shape[1]
    e = edge_index.shape[1]

    n_pad = -(-n // (NS * CHUNK)) * (NS * CHUNK)
    e_pad = -(-e // (NW * CHUNK * NB)) * (NW * CHUNK * NB)
    n_chunks = e_pad // (NW * CHUNK)
    mm = 2 * n_chunks
    m0_32, m1_32 = 80, 80
    m0_16, m1_16 = 80, 80
    assert m0_32 + m1_32 == mm and m0_16 + m1_16 == mm
    padc = max(m0_32, m0_16, m1_32, m1_16)
    grid = n_pad // _BLK

    row = jnp.pad(edge_index[0], (0, e_pad - e))
    col = jnp.pad(edge_index[1], (0, e_pad - e),
                  constant_values=n)
    rowi = jnp.pad(row.reshape(NW * n_chunks, CHUNK), ((0, padc), (0, 0)))
    coli = jnp.pad(col.reshape(NW * n_chunks, CHUNK), ((0, padc), (0, 0)),
                   constant_values=n)
    x_pad = jnp.pad(x, ((0, n_pad - n), (0, 0)))

    invd_col = 3 * hdim + LANES

    hopA = _make_hop_first(n_pad, 2 * hdim, m0_32, m1_32)
    hopC = _make_hop_mid(n_pad, hdim, m0_16, m1_16, hdim, invd_col)
    hopD = _make_hop_relu(n_pad, hdim, m0_16, m1_16)
    hopE = _make_hop_mid(n_pad, hdim, m0_16, m1_16, 0, invd_col)
    degk = _make_deg(n_pad, n_chunks)

    degp = degk(coli)

    w1cat = jnp.concatenate(
        [W1[0:dd], W1[dd:2 * dd], W1[2 * dd:3 * dd]], axis=1)

    pack = pl.pallas_call(
        _proj_body,
        grid=(grid,),
        in_specs=[_pair_spec(LANES), _row_spec(dd), _full_spec(w1cat.shape),
                  _full_spec((1, hdim))],
        out_specs=_row_spec(128),
        out_shape=jax.ShapeDtypeStruct((n_pad, 128), jnp.float32),
    )(degp, x_pad, w1cat, b1.reshape(1, hdim))

    Pab = hopA(pack, rowi, coli)
    Pc = hopC(Pab, pack, rowi, coli)
    Q1, h = hopD(Pab, Pc, pack, rowi, coli)
    Q2 = hopE(Q1, pack, rowi, coli)

    blk2 = 1000
    out = pl.pallas_call(
        _out_body,
        grid=(n // blk2,),
        in_specs=[pl.BlockSpec((blk2, 128), lambda i: (i, 0)),
                  pl.BlockSpec((NC, blk2, hdim), lambda i: (0, i, 0)),
                  pl.BlockSpec((NC, blk2, hdim), lambda i: (0, i, 0)),
                  pl.BlockSpec((blk2, 128), lambda i: (i, 0)),
                  _full_spec(W2.shape), _full_spec((1, dd))],
        out_specs=pl.BlockSpec((blk2, dd), lambda i: (i, 0)),
        out_shape=jax.ShapeDtypeStruct((n, dd), jnp.float32),
    )(h, Q1, Q2, pack, W2, b2.reshape(1, dd))

    return out


# speedup vs baseline: 1.0569x; 1.0005x over previous
import functools

import jax
import jax.numpy as jnp
from jax import lax
from jax.experimental import pallas as pl
from jax.experimental.pallas import tpu as pltpu
from jax.experimental.pallas import tpu_sc as plsc

NC = 2
NS = 16
LANES = 16
NW = NC * NS
CHUNK = 128
NB = 4
NBUF = 2 * NB


def _zero_rows(buf, nrows, d):
    def body(i, _):
        for k in range(d // LANES):
            buf[i, pl.ds(k * LANES, LANES)] = jnp.zeros((LANES,), jnp.float32)
        return 0
    lax.fori_loop(0, nrows, body, 0)


def _hop_edges(ytab_sh, acc_sh, rowi_hbm, coli_hbm, out_hbm, idxr_v, idxc_v,
               rows_v, sems, c, s, m0, m1, mmax, n_pad, d):
    rows_per_sub = n_pad // NS
    grp = rows_per_sub // CHUNK
    m = jnp.where(c == 0, m0, m1)
    base = jnp.where(c == 0, s * m0, NS * m0 + s * m1)
    pltpu.sync_copy(rowi_hbm.at[pl.ds(base, mmax)], idxr_v)
    pltpu.sync_copy(coli_hbm.at[pl.ds(base, mmax)], idxc_v)
    _zero_rows(rows_v.at[0], CHUNK, d)
    for t in range(grp):
        pltpu.sync_copy(rows_v.at[0],
                        acc_sh.at[pl.ds(s * rows_per_sub + t * CHUNK, CHUNK)])
    plsc.subcore_barrier()

    semg = sems[:NBUF]
    semsc = sems[NBUF:]
    for b in range(NB):
        pltpu.async_copy(ytab_sh.at[idxr_v.at[b]], rows_v.at[b], semg[b])

    @pl.loop(0, mmax, step=NBUF)
    def _chunks(g):
        @pl.when(g < m)
        def _():
            for bb in range(NBUF):
                j = g + bb
                b2 = (bb + NB) % NBUF

                pltpu.make_async_copy(
                    ytab_sh.at[idxr_v.at[j]], rows_v.at[bb], semg[bb]).wait()
                pltpu.async_copy(rows_v.at[bb], acc_sh.at[idxc_v.at[j]],
                                 semsc[bb], add=True)

                @pl.when(j >= NB)
                def _():
                    pltpu.make_async_copy(
                        rows_v.at[b2], acc_sh.at[idxc_v.at[j]],
                        semsc[b2]).wait()

                jn = j + NB

                @pl.when(jn < m)
                def _():
                    pltpu.async_copy(ytab_sh.at[idxr_v.at[jn]],
                                     rows_v.at[b2], semg[b2])

    for b in range(NB, NBUF):
        pltpu.make_async_copy(rows_v.at[b], acc_sh.at[idxc_v.at[0]],
                              semsc[b]).wait()

    plsc.subcore_barrier()
    for t in range(grp):
        off = s * rows_per_sub + t * CHUNK
        pltpu.sync_copy(acc_sh.at[pl.ds(off, CHUNK)],
                        out_hbm.at[c, pl.ds(off, CHUNK)])


def _hop_scratch(n_pad, d, mmax):
    return [
        pltpu.VMEM((mmax, CHUNK), jnp.int32),
        pltpu.VMEM((mmax, CHUNK), jnp.int32),
        pltpu.VMEM((NBUF, CHUNK, d), jnp.float32),
        pltpu.VMEM_SHARED((n_pad, d), jnp.float32),
        pltpu.VMEM_SHARED((n_pad, d), jnp.float32),
    ] + [pltpu.SemaphoreType.DMA] * (2 * NBUF)


_MESH = dict(core_axis_name="c", subcore_axis_name="s")


def _make_hop_first(n_pad, d, m0, m1):
    mmax = max(m0, m1)
    rows_per_sub = n_pad // NS

    @functools.partial(
        pl.kernel, mesh=plsc.VectorSubcoreMesh(**_MESH),
        out_type=jax.ShapeDtypeStruct((NC, n_pad, d), jnp.float32),
        compiler_params=pltpu.CompilerParams(use_tc_tiling_on_sc=False),
        scratch_types=_hop_scratch(n_pad, d, mmax))
    def hop(y_hbm, rowi_hbm, coli_hbm, out_hbm, idxr_v, idxc_v, rows_v,
            ytab_sh, acc_sh, *sems):
        c = lax.axis_index("c")
        s = lax.axis_index("s")
        off = s * rows_per_sub
        pltpu.sync_copy(y_hbm.at[pl.ds(off, rows_per_sub), pl.ds(0, d)],
                        ytab_sh.at[pl.ds(off, rows_per_sub)])
        _hop_edges(ytab_sh, acc_sh, rowi_hbm, coli_hbm, out_hbm, idxr_v,
                   idxc_v, rows_v, sems, c, s, m0, m1, mmax, n_pad, d)

    return hop


def _make_hop_mid(n_pad, d, m0, m1, lo_col, invd_col):
    mmax = max(m0, m1)
    rows_per_sub = n_pad // NS
    grp = rows_per_sub // CHUNK

    @functools.partial(
        pl.kernel, mesh=plsc.VectorSubcoreMesh(**_MESH),
        out_type=jax.ShapeDtypeStruct((NC, n_pad, d), jnp.float32),
        compiler_params=pltpu.CompilerParams(use_tc_tiling_on_sc=False),
        scratch_types=[
            pltpu.VMEM((CHUNK, d), jnp.float32),
            pltpu.VMEM((CHUNK, d), jnp.float32),
            pltpu.VMEM((CHUNK, LANES), jnp.float32),
            pltpu.VMEM((CHUNK, d), jnp.float32),
        ] + _hop_scratch(n_pad, d, mmax))
    def hop(p_hbm, invd_hbm, rowi_hbm, coli_hbm, out_hbm, sa, sb, sd, yb,
            idxr_v, idxc_v, rows_v, ytab_sh, acc_sh, *sems):
        c = lax.axis_index("c")
        s = lax.axis_index("s")
        for t in range(grp):
            off = s * rows_per_sub + t * CHUNK
            pltpu.sync_copy(
                p_hbm.at[0, pl.ds(off, CHUNK), pl.ds(lo_col, d)], sa)
            pltpu.sync_copy(
                p_hbm.at[1, pl.ds(off, CHUNK), pl.ds(lo_col, d)], sb)
            pltpu.sync_copy(
                invd_hbm.at[pl.ds(off, CHUNK), pl.ds(invd_col, LANES)], sd)

            def rowbody(i, _):
                for k in range(d // LANES):
                    sl = pl.ds(k * LANES, LANES)
                    yb[i, sl] = (sa[i, sl] + sb[i, sl]) * sd[i, :]
                return 0
            lax.fori_loop(0, CHUNK, rowbody, 0)
            pltpu.sync_copy(yb, ytab_sh.at[pl.ds(off, CHUNK)])
        _hop_edges(ytab_sh, acc_sh, rowi_hbm, coli_hbm, out_hbm, idxr_v,
                   idxc_v, rows_v, sems, c, s, m0, m1, mmax, n_pad, d)

    return hop


def _make_hop_relu(n_pad, d, m0, m1):
    mmax = max(m0, m1)
    rows_per_sub = n_pad // NS
    grp = rows_per_sub // CHUNK

    @functools.partial(
        pl.kernel, mesh=plsc.VectorSubcoreMesh(**_MESH),
        out_type=(jax.ShapeDtypeStruct((NC, n_pad, d), jnp.float32),
                  jax.ShapeDtypeStruct((n_pad, 128), jnp.float32)),
        compiler_params=pltpu.CompilerParams(use_tc_tiling_on_sc=False),
        scratch_types=[
            pltpu.VMEM((CHUNK, d), jnp.float32),
            pltpu.VMEM((CHUNK, d), jnp.float32),
            pltpu.VMEM((CHUNK, d), jnp.float32),
            pltpu.VMEM((CHUNK, d), jnp.float32),
            pltpu.VMEM((CHUNK, d), jnp.float32),
            pltpu.VMEM((CHUNK, LANES), jnp.float32),
            pltpu.VMEM((CHUNK, d), jnp.float32),
            pltpu.VMEM((CHUNK, d), jnp.float32),
        ] + _hop_scratch(n_pad, d, mmax))
    def hop(pab_hbm, pc_hbm, pack_hbm, rowi_hbm, coli_hbm,
            out_hbm, h_hbm, sa, sb, sc0, sc1, sx, sd, yb, hb,
            idxr_v, idxc_v, rows_v, ytab_sh, acc_sh, *sems):
        c = lax.axis_index("c")
        s = lax.axis_index("s")
        for t in range(grp):
            off = s * rows_per_sub + t * CHUNK
            pltpu.sync_copy(pab_hbm.at[0, pl.ds(off, CHUNK), pl.ds(0, d)], sa)
            pltpu.sync_copy(pab_hbm.at[1, pl.ds(off, CHUNK), pl.ds(0, d)], sb)
            pltpu.sync_copy(pc_hbm.at[0, pl.ds(off, CHUNK)], sc0)
            pltpu.sync_copy(pc_hbm.at[1, pl.ds(off, CHUNK)], sc1)
            pltpu.sync_copy(
                pack_hbm.at[pl.ds(off, CHUNK), pl.ds(2 * d, d)], sx)
            pltpu.sync_copy(
                pack_hbm.at[pl.ds(off, CHUNK), pl.ds(3 * d, LANES)], sd)

            def rowbody(i, _):
                for k in range(d // LANES):
                    sl = pl.ds(k * LANES, LANES)
                    z = sa[i, sl] + sb[i, sl] + sc0[i, sl] + sc1[i, sl]
                    h = jnp.maximum(sx[i, sl] + z * sd[i, :], 0.0)
                    hb[i, sl] = h
                    yb[i, sl] = h * sd[i, :]
                return 0
            lax.fori_loop(0, CHUNK, rowbody, 0)
            pltpu.sync_copy(hb, h_hbm.at[pl.ds(off, CHUNK),
                                         pl.ds(0, d)])
            pltpu.sync_copy(yb, ytab_sh.at[pl.ds(off, CHUNK)])
        _hop_edges(ytab_sh, acc_sh, rowi_hbm, coli_hbm, out_hbm, idxr_v,
                   idxc_v, rows_v, sems, c, s, m0, m1, mmax, n_pad, d)

    return hop


def _make_deg(n_pad, n_chunks):
    rows_per_sub = n_pad // NS
    grp = rows_per_sub // CHUNK

    @functools.partial(
        pl.kernel, mesh=plsc.VectorSubcoreMesh(**_MESH),
        out_type=jax.ShapeDtypeStruct((NC, n_pad, LANES), jnp.float32),
        compiler_params=pltpu.CompilerParams(use_tc_tiling_on_sc=False),
        scratch_types=[
            pltpu.VMEM((n_chunks, CHUNK), jnp.int32),
            pltpu.VMEM((CHUNK, LANES), jnp.float32),
            pltpu.VMEM_SHARED((n_pad, LANES), jnp.float32),
        ])
    def deg(coli_hbm, out_hbm, idxc_v, ones_v, acc_sh):
        c = lax.axis_index("c")
        s = lax.axis_index("s")
        wid = c * NS + s
        pltpu.sync_copy(coli_hbm.at[pl.ds(wid * n_chunks, n_chunks)], idxc_v)
        _zero_rows(ones_v, CHUNK, LANES)
        for t in range(grp):
            pltpu.sync_copy(
                ones_v, acc_sh.at[pl.ds(s * rows_per_sub + t * CHUNK, CHUNK)])
        def fill(i, _):
            ones_v[i, pl.ds(0, LANES)] = jnp.ones((LANES,), jnp.float32)
            return 0
        lax.fori_loop(0, CHUNK, fill, 0)
        plsc.subcore_barrier()

        def body(j, _):
            pltpu.sync_copy(ones_v, acc_sh.at[idxc_v.at[j]], add=True)
            return 0
        lax.fori_loop(0, n_chunks, body, 0)
        plsc.subcore_barrier()
        for t in range(grp):
            off = s * rows_per_sub + t * CHUNK
            pltpu.sync_copy(acc_sh.at[pl.ds(off, CHUNK)],
                            out_hbm.at[c, pl.ds(off, CHUNK)])

    return deg


_BLK = 1024


def _proj_a_body(x_ref, w_ref, r_ref):
    r_ref[...] = jnp.dot(x_ref[...], w_ref[...],
                         preferred_element_type=jnp.float32)


def _proj_body(degp_ref, r_ref, b_ref, pack_ref):
    dsum = degp_ref[0] + degp_ref[1]
    pos = dsum > 0
    dinv = jnp.where(pos, lax.rsqrt(dsum), 0.0)
    hdim = b_ref.shape[1]
    r = r_ref[...]
    pack_ref[:, 0:2 * hdim] = r[:, hdim:3 * hdim] * dinv[:, 0:1]
    pack_ref[:, 2 * hdim:3 * hdim] = r[:, 0:hdim] + b_ref[...]
    pack_ref[:, 3 * hdim:3 * hdim + LANES] = dinv
    pack_ref[:, 3 * hdim + LANES:3 * hdim + 2 * LANES] = (
        jnp.where(pos, 1.0 / dsum, 0.0))
    pack_ref[:, 3 * hdim + 2 * LANES:] = jnp.zeros(
        (pack_ref.shape[0], pack_ref.shape[1] - 3 * hdim - 2 * LANES),
        jnp.float32)


def _out_body(h_ref, qa_ref, qb_ref, pack_ref, w_ref, b_ref, out_ref):
    hh = w_ref.shape[0] // 3
    d1 = pack_ref[:, 3 * hh:3 * hh + 1]
    x1 = (qa_ref[0] + qa_ref[1]) * d1
    x2 = (qb_ref[0] + qb_ref[1]) * d1
    hx = jnp.concatenate([h_ref[:, 0:hh], x1, x2], axis=1)
    out_ref[...] = (
        jnp.dot(hx, w_ref[...], preferred_element_type=jnp.float32)
        + b_ref[...])


def _row_spec(d):
    return pl.BlockSpec((_BLK, d), lambda i: (i, 0))


def _pair_spec(d):
    return pl.BlockSpec((NC, _BLK, d), lambda i: (0, i, 0))


def _full_spec(shape):
    return pl.BlockSpec(shape, lambda i: tuple(0 for _ in shape))


def kernel(x, edge_index, W1, b1, W2, b2):
    n, dd = x.shape
    hdim = W1.shape[1]
    e = edge_index.shape[1]

    n_pad = -(-n // (NS * CHUNK)) * (NS * CHUNK)
    e_pad = -(-e // (NW * CHUNK * NB)) * (NW * CHUNK * NB)
    n_chunks = e_pad // (NW * CHUNK)
    mm = 2 * n_chunks
    m0_32, m1_32 = 80, 80
    m0_16, m1_16 = 80, 80
    assert m0_32 + m1_32 == mm and m0_16 + m1_16 == mm
    padc = max(m0_32, m0_16, m1_32, m1_16)
    grid = n_pad // _BLK

    row = jnp.pad(edge_index[0], (0, e_pad - e))
    col = jnp.pad(edge_index[1], (0, e_pad - e),
                  constant_values=n)
    rowi = jnp.pad(row.reshape(NW * n_chunks, CHUNK), ((0, padc), (0, 0)))
    coli = jnp.pad(col.reshape(NW * n_chunks, CHUNK), ((0, padc), (0, 0)),
                   constant_values=n)
    x_pad = jnp.pad(x, ((0, n_pad - n), (0, 0)))

    invd_col = 3 * hdim + LANES

    hopA = _make_hop_first(n_pad, 2 * hdim, m0_32, m1_32)
    hopC = _make_hop_mid(n_pad, hdim, m0_16, m1_16, hdim, invd_col)
    hopD = _make_hop_relu(n_pad, hdim, m0_16, m1_16)
    hopE = _make_hop_mid(n_pad, hdim, m0_16, m1_16, 0, invd_col)
    degk = _make_deg(n_pad, n_chunks)

    degp = degk(coli)

    w1cat = jnp.concatenate(
        [W1[0:dd], W1[dd:2 * dd], W1[2 * dd:3 * dd]], axis=1)

    r = pl.pallas_call(
        _proj_a_body,
        grid=(grid,),
        in_specs=[_row_spec(dd), _full_spec(w1cat.shape)],
        out_specs=_row_spec(3 * hdim),
        out_shape=jax.ShapeDtypeStruct((n_pad, 3 * hdim), jnp.float32),
    )(x_pad, w1cat)

    pack = pl.pallas_call(
        _proj_body,
        grid=(grid,),
        in_specs=[_pair_spec(LANES), _row_spec(3 * hdim),
                  _full_spec((1, hdim))],
        out_specs=_row_spec(128),
        out_shape=jax.ShapeDtypeStruct((n_pad, 128), jnp.float32),
    )(degp, r, b1.reshape(1, hdim))

    Pab = hopA(pack, rowi, coli)
    Pc = hopC(Pab, pack, rowi, coli)
    Q1, h = hopD(Pab, Pc, pack, rowi, coli)
    Q2 = hopE(Q1, pack, rowi, coli)

    blk2 = 1000
    out = pl.pallas_call(
        _out_body,
        grid=(n // blk2,),
        in_specs=[pl.BlockSpec((blk2, 128), lambda i: (i, 0)),
                  pl.BlockSpec((NC, blk2, hdim), lambda i: (0, i, 0)),
                  pl.BlockSpec((NC, blk2, hdim), lambda i: (0, i, 0)),
                  pl.BlockSpec((blk2, 128), lambda i: (i, 0)),
                  _full_spec(W2.shape), _full_spec((1, dd))],
        out_specs=pl.BlockSpec((blk2, dd), lambda i: (i, 0)),
        out_shape=jax.ShapeDtypeStruct((n, dd), jnp.float32),
    )(h, Q1, Q2, pack, W2, b2.reshape(1, dd))

    return out


# speedup vs baseline: 1.1760x; 1.1127x over previous
import functools

import jax
import jax.numpy as jnp
from jax import lax
from jax.experimental import pallas as pl
from jax.experimental.pallas import tpu as pltpu
from jax.experimental.pallas import tpu_sc as plsc

NC = 2
NS = 16
LANES = 16
NW = NC * NS
CHUNK = 128
NB = 4
NBUF = 2 * NB


def _zero_rows(buf, nrows, d):
    def body(i, _):
        for k in range(d // LANES):
            buf[i, pl.ds(k * LANES, LANES)] = jnp.zeros((LANES,), jnp.float32)
        return 0
    lax.fori_loop(0, nrows, body, 0)


def _hop_edges(ytab_sh, acc_sh, rowi_hbm, coli_hbm, out_hbm, idxr_v, idxc_v,
               rows_v, sems, c, s, m0, m1, mmax, n_pad, d):
    rows_per_sub = n_pad // NS
    grp = rows_per_sub // CHUNK
    m = jnp.where(c == 0, m0, m1)
    base = jnp.where(c == 0, s * m0, NS * m0 + s * m1)
    pltpu.sync_copy(rowi_hbm.at[pl.ds(base, mmax)], idxr_v)
    pltpu.sync_copy(coli_hbm.at[pl.ds(base, mmax)], idxc_v)
    _zero_rows(rows_v.at[0], CHUNK, d)
    for t in range(grp):
        pltpu.sync_copy(rows_v.at[0],
                        acc_sh.at[pl.ds(s * rows_per_sub + t * CHUNK, CHUNK)])
    plsc.subcore_barrier()

    semg = sems[:NBUF]
    semsc = sems[NBUF:]
    for b in range(NB):
        pltpu.async_copy(ytab_sh.at[idxr_v.at[b]], rows_v.at[b], semg[b])

    @pl.loop(0, mmax, step=NBUF)
    def _chunks(g):
        @pl.when(g < m)
        def _():
            for bb in range(NBUF):
                j = g + bb
                b2 = (bb + NB) % NBUF

                pltpu.make_async_copy(
                    ytab_sh.at[idxr_v.at[j]], rows_v.at[bb], semg[bb]).wait()
                pltpu.async_copy(rows_v.at[bb], acc_sh.at[idxc_v.at[j]],
                                 semsc[bb], add=True)

                @pl.when(j >= NB)
                def _():
                    pltpu.make_async_copy(
                        rows_v.at[b2], acc_sh.at[idxc_v.at[j]],
                        semsc[b2]).wait()

                jn = j + NB

                @pl.when(jn < m)
                def _():
                    pltpu.async_copy(ytab_sh.at[idxr_v.at[jn]],
                                     rows_v.at[b2], semg[b2])

    for b in range(NB, NBUF):
        pltpu.make_async_copy(rows_v.at[b], acc_sh.at[idxc_v.at[0]],
                              semsc[b]).wait()

    plsc.subcore_barrier()
    for t in range(grp):
        off = s * rows_per_sub + t * CHUNK
        pltpu.sync_copy(acc_sh.at[pl.ds(off, CHUNK)],
                        out_hbm.at[c, pl.ds(off, CHUNK)])


def _hop_scratch(n_pad, d, mmax):
    return [
        pltpu.VMEM((mmax, CHUNK), jnp.int32),
        pltpu.VMEM((mmax, CHUNK), jnp.int32),
        pltpu.VMEM((NBUF, CHUNK, d), jnp.float32),
        pltpu.VMEM_SHARED((n_pad, d), jnp.float32),
        pltpu.VMEM_SHARED((n_pad, d), jnp.float32),
    ] + [pltpu.SemaphoreType.DMA] * (2 * NBUF)


_MESH = dict(core_axis_name="c", subcore_axis_name="s")


def _make_hop_first(n_pad, d, m0, m1):
    mmax = max(m0, m1)
    rows_per_sub = n_pad // NS

    @functools.partial(
        pl.kernel, mesh=plsc.VectorSubcoreMesh(**_MESH),
        out_type=jax.ShapeDtypeStruct((NC, n_pad, d), jnp.float32),
        compiler_params=pltpu.CompilerParams(use_tc_tiling_on_sc=False),
        scratch_types=_hop_scratch(n_pad, d, mmax))
    def hop(y_hbm, rowi_hbm, coli_hbm, out_hbm, idxr_v, idxc_v, rows_v,
            ytab_sh, acc_sh, *sems):
        c = lax.axis_index("c")
        s = lax.axis_index("s")
        off = s * rows_per_sub
        pltpu.sync_copy(y_hbm.at[pl.ds(off, rows_per_sub), pl.ds(0, d)],
                        ytab_sh.at[pl.ds(off, rows_per_sub)])
        _hop_edges(ytab_sh, acc_sh, rowi_hbm, coli_hbm, out_hbm, idxr_v,
                   idxc_v, rows_v, sems, c, s, m0, m1, mmax, n_pad, d)

    return hop


def _make_hop_mid(n_pad, d, m0, m1, lo_col, invd_col):
    mmax = max(m0, m1)
    rows_per_sub = n_pad // NS
    grp = rows_per_sub // CHUNK

    @functools.partial(
        pl.kernel, mesh=plsc.VectorSubcoreMesh(**_MESH),
        out_type=jax.ShapeDtypeStruct((NC, n_pad, d), jnp.float32),
        compiler_params=pltpu.CompilerParams(use_tc_tiling_on_sc=False),
        scratch_types=[
            pltpu.VMEM((CHUNK, d), jnp.float32),
            pltpu.VMEM((CHUNK, d), jnp.float32),
            pltpu.VMEM((CHUNK, LANES), jnp.float32),
            pltpu.VMEM((CHUNK, d), jnp.float32),
        ] + _hop_scratch(n_pad, d, mmax))
    def hop(p_hbm, invd_hbm, rowi_hbm, coli_hbm, out_hbm, sa, sb, sd, yb,
            idxr_v, idxc_v, rows_v, ytab_sh, acc_sh, *sems):
        c = lax.axis_index("c")
        s = lax.axis_index("s")
        for t in range(grp):
            off = s * rows_per_sub + t * CHUNK
            cps = [
                pltpu.async_copy(
                    p_hbm.at[0, pl.ds(off, CHUNK), pl.ds(lo_col, d)], sa,
                    sems[0]),
                pltpu.async_copy(
                    p_hbm.at[1, pl.ds(off, CHUNK), pl.ds(lo_col, d)], sb,
                    sems[1]),
                pltpu.async_copy(
                    invd_hbm.at[pl.ds(off, CHUNK), pl.ds(invd_col, LANES)],
                    sd, sems[2]),
            ]
            for cp in cps:
                cp.wait()

            def rowbody(i, _):
                for k in range(d // LANES):
                    sl = pl.ds(k * LANES, LANES)
                    yb[i, sl] = (sa[i, sl] + sb[i, sl]) * sd[i, :]
                return 0
            lax.fori_loop(0, CHUNK, rowbody, 0)
            pltpu.sync_copy(yb, ytab_sh.at[pl.ds(off, CHUNK)])
        _hop_edges(ytab_sh, acc_sh, rowi_hbm, coli_hbm, out_hbm, idxr_v,
                   idxc_v, rows_v, sems, c, s, m0, m1, mmax, n_pad, d)

    return hop


def _make_hop_relu(n_pad, d, m0, m1):
    mmax = max(m0, m1)
    rows_per_sub = n_pad // NS
    grp = rows_per_sub // CHUNK

    @functools.partial(
        pl.kernel, mesh=plsc.VectorSubcoreMesh(**_MESH),
        out_type=(jax.ShapeDtypeStruct((NC, n_pad, d), jnp.float32),
                  jax.ShapeDtypeStruct((n_pad, 128), jnp.float32)),
        compiler_params=pltpu.CompilerParams(use_tc_tiling_on_sc=False),
        scratch_types=[
            pltpu.VMEM((CHUNK, d), jnp.float32),
            pltpu.VMEM((CHUNK, d), jnp.float32),
            pltpu.VMEM((CHUNK, d), jnp.float32),
            pltpu.VMEM((CHUNK, d), jnp.float32),
            pltpu.VMEM((CHUNK, d), jnp.float32),
            pltpu.VMEM((CHUNK, LANES), jnp.float32),
            pltpu.VMEM((CHUNK, d), jnp.float32),
            pltpu.VMEM((CHUNK, d), jnp.float32),
        ] + _hop_scratch(n_pad, d, mmax))
    def hop(pab_hbm, pc_hbm, pack_hbm, rowi_hbm, coli_hbm,
            out_hbm, h_hbm, sa, sb, sc0, sc1, sx, sd, yb, hb,
            idxr_v, idxc_v, rows_v, ytab_sh, acc_sh, *sems):
        c = lax.axis_index("c")
        s = lax.axis_index("s")
        for t in range(grp):
            off = s * rows_per_sub + t * CHUNK
            cps = [
                pltpu.async_copy(
                    pab_hbm.at[0, pl.ds(off, CHUNK), pl.ds(0, d)], sa,
                    sems[0]),
                pltpu.async_copy(
                    pab_hbm.at[1, pl.ds(off, CHUNK), pl.ds(0, d)], sb,
                    sems[1]),
                pltpu.async_copy(pc_hbm.at[0, pl.ds(off, CHUNK)], sc0,
                                 sems[2]),
                pltpu.async_copy(pc_hbm.at[1, pl.ds(off, CHUNK)], sc1,
                                 sems[3]),
                pltpu.async_copy(
                    pack_hbm.at[pl.ds(off, CHUNK), pl.ds(2 * d, d)], sx,
                    sems[4]),
                pltpu.async_copy(
                    pack_hbm.at[pl.ds(off, CHUNK), pl.ds(3 * d, LANES)], sd,
                    sems[5]),
            ]
            for cp in cps:
                cp.wait()

            def rowbody(i, _):
                for k in range(d // LANES):
                    sl = pl.ds(k * LANES, LANES)
                    z = sa[i, sl] + sb[i, sl] + sc0[i, sl] + sc1[i, sl]
                    h = jnp.maximum(sx[i, sl] + z * sd[i, :], 0.0)
                    hb[i, sl] = h
                    yb[i, sl] = h * sd[i, :]
                return 0
            lax.fori_loop(0, CHUNK, rowbody, 0)
            pltpu.sync_copy(hb, h_hbm.at[pl.ds(off, CHUNK),
                                         pl.ds(0, d)])
            pltpu.sync_copy(yb, ytab_sh.at[pl.ds(off, CHUNK)])
        _hop_edges(ytab_sh, acc_sh, rowi_hbm, coli_hbm, out_hbm, idxr_v,
                   idxc_v, rows_v, sems, c, s, m0, m1, mmax, n_pad, d)

    return hop


def _make_deg(n_pad, n_chunks):
    rows_per_sub = n_pad // NS
    grp = rows_per_sub // CHUNK

    @functools.partial(
        pl.kernel, mesh=plsc.VectorSubcoreMesh(**_MESH),
        out_type=jax.ShapeDtypeStruct((NC, n_pad, LANES), jnp.float32),
        compiler_params=pltpu.CompilerParams(use_tc_tiling_on_sc=False),
        scratch_types=[
            pltpu.VMEM((n_chunks, CHUNK), jnp.int32),
            pltpu.VMEM((CHUNK, LANES), jnp.float32),
            pltpu.VMEM_SHARED((n_pad, LANES), jnp.float32),
        ])
    def deg(coli_hbm, out_hbm, idxc_v, ones_v, acc_sh):
        c = lax.axis_index("c")
        s = lax.axis_index("s")
        wid = c * NS + s
        pltpu.sync_copy(coli_hbm.at[pl.ds(wid * n_chunks, n_chunks)], idxc_v)
        _zero_rows(ones_v, CHUNK, LANES)
        for t in range(grp):
            pltpu.sync_copy(
                ones_v, acc_sh.at[pl.ds(s * rows_per_sub + t * CHUNK, CHUNK)])
        def fill(i, _):
            ones_v[i, pl.ds(0, LANES)] = jnp.ones((LANES,), jnp.float32)
            return 0
        lax.fori_loop(0, CHUNK, fill, 0)
        plsc.subcore_barrier()

        def body(j, _):
            pltpu.sync_copy(ones_v, acc_sh.at[idxc_v.at[j]], add=True)
            return 0
        lax.fori_loop(0, n_chunks, body, 0)
        plsc.subcore_barrier()
        for t in range(grp):
            off = s * rows_per_sub + t * CHUNK
            pltpu.sync_copy(acc_sh.at[pl.ds(off, CHUNK)],
                            out_hbm.at[c, pl.ds(off, CHUNK)])

    return deg


_BLK = 1024


def _proj_a_body(x_ref, w_ref, r_ref):
    r_ref[...] = jnp.dot(x_ref[...], w_ref[...],
                         preferred_element_type=jnp.float32)


def _proj_body(degp_ref, r_ref, b_ref, pack_ref):
    dsum = degp_ref[0] + degp_ref[1]
    pos = dsum > 0
    dinv = jnp.where(pos, lax.rsqrt(dsum), 0.0)
    hdim = b_ref.shape[1]
    r = r_ref[...]
    pack_ref[:, 0:2 * hdim] = r[:, hdim:3 * hdim] * dinv[:, 0:1]
    pack_ref[:, 2 * hdim:3 * hdim] = r[:, 0:hdim] + b_ref[...]
    pack_ref[:, 3 * hdim:3 * hdim + LANES] = dinv
    pack_ref[:, 3 * hdim + LANES:3 * hdim + 2 * LANES] = (
        jnp.where(pos, 1.0 / dsum, 0.0))
    pack_ref[:, 3 * hdim + 2 * LANES:] = jnp.zeros(
        (pack_ref.shape[0], pack_ref.shape[1] - 3 * hdim - 2 * LANES),
        jnp.float32)


def _out_body(h_ref, qa_ref, qb_ref, pack_ref, w_ref, b_ref, out_ref):
    hh = w_ref.shape[0] // 3
    d1 = pack_ref[:, 3 * hh:3 * hh + 1]
    x1 = (qa_ref[0] + qa_ref[1]) * d1
    x2 = (qb_ref[0] + qb_ref[1]) * d1
    hx = jnp.concatenate([h_ref[:, 0:hh], x1, x2], axis=1)
    out_ref[...] = (
        jnp.dot(hx, w_ref[...], preferred_element_type=jnp.float32)
        + b_ref[...])


def _row_spec(d):
    return pl.BlockSpec((_BLK, d), lambda i: (i, 0))


def _pair_spec(d):
    return pl.BlockSpec((NC, _BLK, d), lambda i: (0, i, 0))


def _full_spec(shape):
    return pl.BlockSpec(shape, lambda i: tuple(0 for _ in shape))


def kernel(x, edge_index, W1, b1, W2, b2):
    n, dd = x.shape
    hdim = W1.shape[1]
    e = edge_index.shape[1]

    n_pad = -(-n // (NS * CHUNK)) * (NS * CHUNK)
    e_pad = -(-e // (NW * CHUNK * NB)) * (NW * CHUNK * NB)
    n_chunks = e_pad // (NW * CHUNK)
    mm = 2 * n_chunks
    m0_32, m1_32 = 80, 80
    m0_16, m1_16 = 80, 80
    assert m0_32 + m1_32 == mm and m0_16 + m1_16 == mm
    padc = max(m0_32, m0_16, m1_32, m1_16)
    grid = n_pad // _BLK

    row = jnp.pad(edge_index[0], (0, e_pad - e))
    col = jnp.pad(edge_index[1], (0, e_pad - e),
                  constant_values=n)
    rowi = jnp.pad(row.reshape(NW * n_chunks, CHUNK), ((0, padc), (0, 0)))
    coli = jnp.pad(col.reshape(NW * n_chunks, CHUNK), ((0, padc), (0, 0)),
                   constant_values=n)
    x_pad = jnp.pad(x, ((0, n_pad - n), (0, 0)))

    invd_col = 3 * hdim + LANES

    hopA = _make_hop_first(n_pad, 2 * hdim, m0_32, m1_32)
    hopC = _make_hop_mid(n_pad, hdim, m0_16, m1_16, hdim, invd_col)
    hopD = _make_hop_relu(n_pad, hdim, m0_16, m1_16)
    hopE = _make_hop_mid(n_pad, hdim, m0_16, m1_16, 0, invd_col)
    degk = _make_deg(n_pad, n_chunks)

    degp = degk(coli)

    w1cat = jnp.concatenate(
        [W1[0:dd], W1[dd:2 * dd], W1[2 * dd:3 * dd]], axis=1)

    r = pl.pallas_call(
        _proj_a_body,
        grid=(grid,),
        in_specs=[_row_spec(dd), _full_spec(w1cat.shape)],
        out_specs=_row_spec(3 * hdim),
        out_shape=jax.ShapeDtypeStruct((n_pad, 3 * hdim), jnp.float32),
    )(x_pad, w1cat)

    pack = pl.pallas_call(
        _proj_body,
        grid=(grid,),
        in_specs=[_pair_spec(LANES), _row_spec(3 * hdim),
                  _full_spec((1, hdim))],
        out_specs=_row_spec(128),
        out_shape=jax.ShapeDtypeStruct((n_pad, 128), jnp.float32),
    )(degp, r, b1.reshape(1, hdim))

    Pab = hopA(pack, rowi, coli)
    Pc = hopC(Pab, pack, rowi, coli)
    Q1, h = hopD(Pab, Pc, pack, rowi, coli)
    Q2 = hopE(Q1, pack, rowi, coli)

    blk2 = 1000
    out = pl.pallas_call(
        _out_body,
        grid=(n // blk2,),
        in_specs=[pl.BlockSpec((blk2, 128), lambda i: (i, 0)),
                  pl.BlockSpec((NC, blk2, hdim), lambda i: (0, i, 0)),
                  pl.BlockSpec((NC, blk2, hdim), lambda i: (0, i, 0)),
                  pl.BlockSpec((blk2, 128), lambda i: (i, 0)),
                  _full_spec(W2.shape), _full_spec((1, dd))],
        out_specs=pl.BlockSpec((blk2, dd), lambda i: (i, 0)),
        out_shape=jax.ShapeDtypeStruct((n, dd), jnp.float32),
    )(h, Q1, Q2, pack, W2, b2.reshape(1, dd))

    return out
